# baseline (speedup 1.0000x reference)
import dataclasses

import jax
import jax.numpy as jnp
from jax import lax
from jax.experimental import pallas as pl
from jax.experimental.pallas import tpu as pltpu
from jax.experimental.pallas import tpu_sc as plsc

N = 10000
E = 160000
D_IN = 256
HID = 256
HEADS = 4

NC, NS, LANES = 2, 16, 16
NW = NC * NS
NP = 10240
NPS = NP // NS
BN = 1000

KA = 40
KC = 40
KE1 = 80
KE2 = 16

_MESH = plsc.VectorSubcoreMesh(
    core_axis_name="c", subcore_axis_name="s", num_cores=NC, num_subcores=NS)

f32 = jnp.float32
i32 = jnp.int32

_SC_PARAMS = pltpu.CompilerParams()
if "needs_layout_passes" in pltpu.CompilerParams.__dataclass_fields__:
    _SC_PARAMS = dataclasses.replace(_SC_PARAMS, needs_layout_passes=False)


def _fill_zero(buf):
    rows, cols = buf.shape
    z = jnp.zeros((LANES,), f32)
    for r in range(rows):
        for g in range(cols // LANES):
            buf.at[r][pl.ds(g * LANES, LANES)] = z


def _zero_spmem(sp, zbuf, sid):
    zr = zbuf.shape[0]
    reps = NPS // zr

    @pl.loop(0, reps)
    def _(i):
        pltpu.sync_copy(zbuf, sp.at[pl.ds(sid * NPS + i * zr, zr)])


def _hist_body(dst_hbm, out_hbm, idxv, onesv, zbuf, hist_sp):
    cid = lax.axis_index("c")
    sid = lax.axis_index("s")
    w = sid * NC + cid
    ii = lax.iota(i32, LANES)
    row1 = jnp.where(ii == 0, 1.0, 0.0).astype(f32)
    _fill_zero(onesv)
    for r in range(KA):
        onesv.at[r][pl.ds(0, LANES)] = row1
    _fill_zero(zbuf)
    _zero_spmem(hist_sp, zbuf, sid)
    plsc.subcore_barrier()

    @pl.loop(0, (E // KA) // NW)
    def _(j):
        base = (w + j * NW) * KA
        pltpu.sync_copy(dst_hbm.at[pl.ds(base, KA)], idxv)
        pltpu.sync_copy(onesv, hist_sp.at[idxv], add=True)

    plsc.subcore_barrier()
    pltpu.sync_copy(hist_sp.at[pl.ds(sid * NPS, NPS)],
                    out_hbm.at[cid, pl.ds(sid * NPS, NPS)])


def _sc_hist(dst):
    return pl.kernel(
        _hist_body,
        out_type=jax.ShapeDtypeStruct((NC, NP, 128), f32),
        mesh=_MESH,
        scratch_types=[
            pltpu.VMEM((KA,), i32),
            pltpu.VMEM((KA, 128), f32),
            pltpu.VMEM((32, 128), f32),
            pltpu.VMEM_SHARED((NP, 128), f32),
        ],
    )(dst)


def _agg_body(tab_hbm, srcs_hbm, dst_hbm, out_hbm, gidx, sidx, rows, zbuf,
              acc_sp):
    cid = lax.axis_index("c")
    sid = lax.axis_index("s")
    _fill_zero(zbuf)
    _zero_spmem(acc_sp, zbuf, sid)
    plsc.subcore_barrier()

    @pl.loop(0, (E // KC) // NS)
    def _(j):
        base = (sid + j * NS) * KC
        pltpu.sync_copy(srcs_hbm.at[pl.ds(cid * E + base, KC)], gidx)
        pltpu.sync_copy(dst_hbm.at[pl.ds(base, KC)], sidx)
        pltpu.sync_copy(tab_hbm.at[gidx], rows)
        pltpu.sync_copy(rows, acc_sp.at[sidx], add=True)

    plsc.subcore_barrier()
    pltpu.sync_copy(acc_sp.at[pl.ds(sid * NPS, NPS)],
                    out_hbm.at[cid, pl.ds(sid * NPS, NPS)])


def _sc_agg(tab2, src2, dst):
    return pl.kernel(
        _agg_body,
        out_type=jax.ShapeDtypeStruct((NC, NP, 128), f32),
        mesh=_MESH,
        scratch_types=[
            pltpu.VMEM((KC,), i32),
            pltpu.VMEM((KC,), i32),
            pltpu.VMEM((KC, 128), f32),
            pltpu.VMEM((32, 128), f32),
            pltpu.VMEM_SHARED((NP, 128), f32),
        ],
    )(tab2, src2, dst)


def _gat1_body(src_hbm, dst_hbm, asad_hbm, mrep_hbm, ee_hbm, den_hbm,
               mrep_v, srcb, dstb, asb, adb, st80, eeb, zbuf, den_sp):
    cid = lax.axis_index("c")
    sid = lax.axis_index("s")
    pltpu.sync_copy(mrep_hbm, mrep_v)
    _fill_zero(st80)
    _fill_zero(zbuf)
    _zero_spmem(den_sp, zbuf, sid)
    plsc.subcore_barrier()

    io = lax.iota(i32, LANES)

    @pl.loop(0, (E // KE1) // NS)
    def _(j):
        base = (sid + j * NS) * KE1
        pltpu.sync_copy(src_hbm.at[pl.ds(base, KE1)], srcb)
        pltpu.sync_copy(dst_hbm.at[pl.ds(base, KE1)], dstb)
        pltpu.sync_copy(asad_hbm.at[srcb], asb)
        pltpu.sync_copy(asad_hbm.at[dstb], adb)
        for i in range(KE1 // LANES):
            r16 = i * LANES + io
            for hd in range(HEADS):
                hv = jnp.full((LANES,), hd, i32)
                av = plsc.load_gather(asb, [r16, hv])
                bv = plsc.load_gather(adb, [r16, hv + HEADS])
                ev = av + bv
                lr = jnp.maximum(ev, 0.2 * ev)
                fv = jnp.exp(lr - mrep_v.at[hd][pl.ds(0, LANES)])
                plsc.store_scatter(st80, [r16, hv], fv)
                plsc.store_scatter(eeb, [r16 * HEADS + hv], fv)
        pltpu.sync_copy(st80, den_sp.at[dstb], add=True)
        pltpu.sync_copy(eeb, ee_hbm.at[pl.ds(base * HEADS, KE1 * HEADS)])

    plsc.subcore_barrier()
    pltpu.sync_copy(den_sp.at[pl.ds(sid * NPS, NPS)],
                    den_hbm.at[cid, pl.ds(sid * NPS, NPS)])


def _sc_gat1(src, dst, asad128, mrep):
    return pl.kernel(
        _gat1_body,
        out_type=[jax.ShapeDtypeStruct((E * HEADS,), f32),
                  jax.ShapeDtypeStruct((NC, NP, 128), f32)],
        mesh=_MESH,
        scratch_types=[
            pltpu.VMEM((HEADS, 16), f32),
            pltpu.VMEM((KE1,), i32),
            pltpu.VMEM((KE1,), i32),
            pltpu.VMEM((KE1, 128), f32),
            pltpu.VMEM((KE1, 128), f32),
            pltpu.VMEM((KE1, 128), f32),
            pltpu.VMEM((KE1 * HEADS,), f32),
            pltpu.VMEM((32, 128), f32),
            pltpu.VMEM_SHARED((NP, 128), f32),
        ],
        compiler_params=_SC_PARAMS,
    )(src, dst, asad128, mrep)


def _gat2_body(hh2_hbm, srcs_hbm, dst_hbm, ee_hbm, invd_hbm, out_hbm,
               srcb, dstb, eeb, wb, ivb, rows, outst, zbuf, gacc_sp):
    cid = lax.axis_index("c")
    sid = lax.axis_index("s")
    _fill_zero(zbuf)
    _zero_spmem(gacc_sp, zbuf, sid)
    plsc.subcore_barrier()

    io = lax.iota(i32, LANES)

    @pl.loop(0, (E // KE2) // NS)
    def _(j):
        base = (sid + j * NS) * KE2
        pltpu.sync_copy(srcs_hbm.at[pl.ds(cid * E + base, KE2)], srcb)
        pltpu.sync_copy(dst_hbm.at[pl.ds(base, KE2)], dstb)
        pltpu.sync_copy(ee_hbm.at[pl.ds(base * HEADS, KE2 * HEADS)], eeb)
        pltpu.sync_copy(invd_hbm.at[dstb], ivb)
        pltpu.sync_copy(hh2_hbm.at[srcb], rows)
        for hd in range(HEADS):
            hv = jnp.full((LANES,), hd, i32)
            ee = plsc.load_gather(eeb, [io * HEADS + hv])
            iv = plsc.load_gather(ivb, [io, hv])
            plsc.store_scatter(wb, [io * HEADS + hv], ee * iv)

        @pl.loop(0, KE2)
        def _(r):
            rv = jnp.zeros((LANES,), i32) + r * HEADS
            acc = [jnp.zeros((LANES,), f32) for _ in range(8)]
            for hd in range(HEADS):
                hv = jnp.full((LANES,), hd, i32)
                wv = plsc.load_gather(wb, [rv + hv])
                for g in range(8):
                    acc[g] = acc[g] + wv * rows[r, pl.ds(
                        hd * 128 + g * LANES, LANES)]
            for g in range(8):
                outst[r, pl.ds(g * LANES, LANES)] = acc[g]

        pltpu.sync_copy(outst, gacc_sp.at[dstb], add=True)

    plsc.subcore_barrier()
    pltpu.sync_copy(gacc_sp.at[pl.ds(sid * NPS, NPS)],
                    out_hbm.at[cid, pl.ds(sid * NPS, NPS)])


def _sc_gat2(hh2, src2, dst, ee4, invd128):
    return pl.kernel(
        _gat2_body,
        out_type=jax.ShapeDtypeStruct((NC, NP, 128), f32),
        mesh=_MESH,
        scratch_types=[
            pltpu.VMEM((KE2,), i32),
            pltpu.VMEM((KE2,), i32),
            pltpu.VMEM((KE2 * HEADS,), f32),
            pltpu.VMEM((KE2 * HEADS,), f32),
            pltpu.VMEM((KE2, 128), f32),
            pltpu.VMEM((KE2, 512), f32),
            pltpu.VMEM((KE2, 128), f32),
            pltpu.VMEM((16, 128), f32),
            pltpu.VMEM_SHARED((NP, 128), f32),
        ],
        compiler_params=_SC_PARAMS,
    )(hh2, src2, dst, ee4, invd128)


def _mm_body(x_ref, w_ref, o_ref):
    o_ref[...] = jnp.dot(x_ref[...], w_ref[...],
                         preferred_element_type=f32)


def _tc_matmul(x, w):
    m, k = x.shape
    _, n = w.shape
    return pl.pallas_call(
        _mm_body,
        grid=(m // BN,),
        in_specs=[pl.BlockSpec((BN, k), lambda i: (i, 0)),
                  pl.BlockSpec((k, n), lambda i: (0, 0))],
        out_specs=pl.BlockSpec((BN, n), lambda i: (i, 0)),
        out_shape=jax.ShapeDtypeStruct((m, n), f32),
    )(x, w)


def _prescale_body(hx_ref, h0_ref, h1_ref, hp_ref, dinv_ref, degn1_ref):
    degn = h0_ref[...] + h1_ref[...]
    deg = degn + 1.0
    dinv = lax.rsqrt(deg)
    hp_ref[...] = hx_ref[...] * dinv
    dinv_ref[...] = dinv
    degn1_ref[...] = jnp.maximum(degn, 1.0)


def _tc_prescale(hx, h0, h1):
    return pl.pallas_call(
        _prescale_body,
        grid=(N // BN,),
        in_specs=[pl.BlockSpec((BN, HID), lambda i: (i, 0)),
                  pl.BlockSpec((BN, 1), lambda i: (i, 0)),
                  pl.BlockSpec((BN, 1), lambda i: (i, 0))],
        out_specs=[pl.BlockSpec((BN, HID), lambda i: (i, 0)),
                   pl.BlockSpec((BN, 1), lambda i: (i, 0)),
                   pl.BlockSpec((BN, 1), lambda i: (i, 0))],
        out_shape=[jax.ShapeDtypeStruct((N, HID), f32),
                   jax.ShapeDtypeStruct((N, 1), f32),
                   jax.ShapeDtypeStruct((N, 1), f32)],
    )(hx, h0, h1)


def _gcn_gat_body(agg_ref, hp_ref, dinv_ref, bg_ref, wgat_ref, amat_ref,
                  hh_ref, asad_ref, macc_ref):
    i = pl.program_id(0)
    h = jnp.maximum(dinv_ref[...] * (agg_ref[...] + hp_ref[...])
                    + bg_ref[...], 0.0)
    hh = jnp.dot(h, wgat_ref[...], preferred_element_type=f32)
    hh_ref[...] = hh
    asad = jnp.dot(hh, amat_ref[...], preferred_element_type=f32)
    asad_ref[...] = jnp.concatenate(
        [asad, jnp.zeros((asad.shape[0], 120), f32)], axis=1)
    mx = jnp.max(asad, axis=0, keepdims=True)
    mx8 = jnp.broadcast_to(mx, (8, 8))
    prev = jnp.where(i == 0, jnp.full((8, 8), -jnp.inf, f32), macc_ref[...])
    macc_ref[...] = jnp.maximum(prev, mx8)


def _tc_gcn_gat(aggcat, hp, dinv, b_gcn, W_gat, Amat):
    return pl.pallas_call(
        _gcn_gat_body,
        grid=(N // BN,),
        in_specs=[pl.BlockSpec((BN, HID), lambda i: (i, 0)),
                  pl.BlockSpec((BN, HID), lambda i: (i, 0)),
                  pl.BlockSpec((BN, 1), lambda i: (i, 0)),
                  pl.BlockSpec((1, HID), lambda i: (0, 0)),
                  pl.BlockSpec((HID, HEADS * HID), lambda i: (0, 0)),
                  pl.BlockSpec((HEADS * HID, 8), lambda i: (0, 0))],
        out_specs=[pl.BlockSpec((BN, HEADS * HID), lambda i: (i, 0)),
                   pl.BlockSpec((BN, 128), lambda i: (i, 0)),
                   pl.BlockSpec((8, 8), lambda i: (0, 0))],
        out_shape=[jax.ShapeDtypeStruct((N, HEADS * HID), f32),
                   jax.ShapeDtypeStruct((N, 128), f32),
                   jax.ShapeDtypeStruct((8, 8), f32)],
    )(aggcat, hp, dinv, b_gcn, W_gat, Amat)


def _denom_body(den_ref, asad_ref, m4_ref, hh_ref, invd_ref, gself_ref):
    den = 0.5 * (den_ref[0, :, :HEADS] + den_ref[1, :, :HEADS])
    es = asad_ref[:, :HEADS] + asad_ref[:, HEADS:]
    lr = jnp.maximum(es, 0.2 * es)
    se = jnp.exp(lr - m4_ref[...])
    dtot = den + se
    invd = 1.0 / (4.0 * dtot)
    invd_ref[...] = jnp.concatenate(
        [invd, jnp.zeros((invd.shape[0], 124), f32)], axis=1)
    acc = jnp.zeros_like(gself_ref)
    for hd in range(HEADS):
        coef = se[:, hd:hd + 1] * invd[:, hd:hd + 1]
        acc = acc + coef * hh_ref[:, hd * HID:(hd + 1) * HID]
    gself_ref[...] = acc


def _tc_denom(denp, asad, M4r, hh):
    return pl.pallas_call(
        _denom_body,
        grid=(N // BN,),
        in_specs=[pl.BlockSpec((NC, BN, 128), lambda i: (0, i, 0)),
                  pl.BlockSpec((BN, 8), lambda i: (i, 0)),
                  pl.BlockSpec((1, HEADS), lambda i: (0, 0)),
                  pl.BlockSpec((BN, HEADS * HID), lambda i: (i, 0))],
        out_specs=[pl.BlockSpec((BN, 128), lambda i: (i, 0)),
                   pl.BlockSpec((BN, HID), lambda i: (i, 0))],
        out_shape=[jax.ShapeDtypeStruct((N, 128), f32),
                   jax.ShapeDtypeStruct((N, HID), f32)],
    )(denp, asad, M4r, hh)


def _gat_fin_body(graw_ref, gself_ref, bgat_ref, g_ref):
    g_ref[...] = jnp.maximum(graw_ref[...] + gself_ref[...] + bgat_ref[...],
                             0.0)


def _tc_gat_fin(grawcat, gself, b_gat):
    return pl.pallas_call(
        _gat_fin_body,
        grid=(N // BN,),
        in_specs=[pl.BlockSpec((BN, HID), lambda i: (i, 0)),
                  pl.BlockSpec((BN, HID), lambda i: (i, 0)),
                  pl.BlockSpec((1, HID), lambda i: (0, 0))],
        out_specs=pl.BlockSpec((BN, HID), lambda i: (i, 0)),
        out_shape=jax.ShapeDtypeStruct((N, HID), f32),
    )(grawcat, gself, b_gat)


def _head_body(nsum_ref, degn1_ref, g_ref, wl_ref, wr_ref, bs_ref,
               wc1_ref, bc1_ref, wc2_ref, bc2_ref, o_ref):
    neigh = nsum_ref[...] / degn1_ref[...]
    s = (jnp.dot(neigh, wl_ref[...], preferred_element_type=f32)
         + jnp.dot(g_ref[...], wr_ref[...], preferred_element_type=f32)
         + bs_ref[...])
    c = jnp.maximum(jnp.dot(s, wc1_ref[...], preferred_element_type=f32)
                    + bc1_ref[...], 0.0)
    logits = jnp.dot(c, wc2_ref[...], preferred_element_type=f32) + bc2_ref[...]
    o_ref[...] = jax.nn.sigmoid(logits)


def _tc_head(nsumcat, degn1, g, W_sage_l, W_sage_r, bs, W_c1, bc1, W_c2p, bc2):
    return pl.pallas_call(
        _head_body,
        grid=(N // BN,),
        in_specs=[pl.BlockSpec((BN, HID), lambda i: (i, 0)),
                  pl.BlockSpec((BN, 1), lambda i: (i, 0)),
                  pl.BlockSpec((BN, HID), lambda i: (i, 0)),
                  pl.BlockSpec((HID, HID), lambda i: (0, 0)),
                  pl.BlockSpec((HID, HID), lambda i: (0, 0)),
                  pl.BlockSpec((1, HID), lambda i: (0, 0)),
                  pl.BlockSpec((HID, HID // 2), lambda i: (0, 0)),
                  pl.BlockSpec((1, HID // 2), lambda i: (0, 0)),
                  pl.BlockSpec((HID // 2, 8), lambda i: (0, 0)),
                  pl.BlockSpec((1, 8), lambda i: (0, 0))],
        out_specs=pl.BlockSpec((BN, 8), lambda i: (i, 0)),
        out_shape=jax.ShapeDtypeStruct((N, 8), f32),
    )(nsumcat, degn1, g, W_sage_l, W_sage_r, bs, W_c1, bc1, W_c2p, bc2)


def kernel(x, edge_index, W_gcn, b_gcn, W_gat, att_src, att_dst, b_gat,
           W_sage_l, W_sage_r, b_sage, W_c1, b_c1, W_c2, b_c2):
    src = edge_index[0]
    dst = edge_index[1]
    src2 = jnp.concatenate([src, src + N])

    hist = _sc_hist(dst)
    hx = _tc_matmul(x, W_gcn)
    h0 = hist[0, :N, :1]
    h1 = hist[1, :N, :1]
    hp, dinv, degn1 = _tc_prescale(hx, h0, h1)
    hp2 = jnp.concatenate([hp[:, :128], hp[:, 128:]], axis=0)
    agg = _sc_agg(hp2, src2, dst)
    aggcat = jnp.concatenate([agg[0, :N], agg[1, :N]], axis=1)

    eye = jnp.eye(HEADS, dtype=f32)
    As = (att_src[:, :, None] * eye[:, None, :]).reshape(HEADS * HID, HEADS)
    Ad = (att_dst[:, :, None] * eye[:, None, :]).reshape(HEADS * HID, HEADS)
    Amat = jnp.concatenate([As, Ad], axis=1)

    hh, asad, macc = _tc_gcn_gat(aggcat, hp, dinv, b_gcn.reshape(1, HID),
                                 W_gat, Amat)
    ms = macc[0, :HEADS]
    md = macc[0, HEADS:]
    msum = ms + md
    M4 = jnp.maximum(msum, 0.2 * msum)
    mrep = jnp.broadcast_to(M4[:, None], (HEADS, 16))

    ee4, denp = _sc_gat1(src, dst, asad, mrep)
    invd128, gself = _tc_denom(denp, asad[:, :8], M4.reshape(1, HEADS), hh)

    hh4 = hh.reshape(N, HEADS, HID)
    hh2 = jnp.concatenate([hh4[:, :, :128].reshape(N, HEADS * 128),
                           hh4[:, :, 128:].reshape(N, HEADS * 128)], axis=0)
    graw = _sc_gat2(hh2, src2, dst, ee4, invd128)
    grawcat = jnp.concatenate([graw[0, :N], graw[1, :N]], axis=1)
    g = _tc_gat_fin(grawcat, gself, b_gat.reshape(1, HID))

    g2 = jnp.concatenate([g[:, :128], g[:, 128:]], axis=0)
    nsum = _sc_agg(g2, src2, dst)
    nsumcat = jnp.concatenate([nsum[0, :N], nsum[1, :N]], axis=1)

    W_c2p = jnp.concatenate([W_c2, jnp.zeros((HID // 2, 7), f32)], axis=1)
    bc2p = jnp.concatenate([b_c2, jnp.zeros((7,), f32)]).reshape(1, 8)
    out8 = _tc_head(nsumcat, degn1, g, W_sage_l, W_sage_r,
                    b_sage.reshape(1, HID), W_c1, b_c1.reshape(1, HID // 2),
                    W_c2p, bc2p)
    return out8[:, :1]


# speedup vs baseline: 1.2450x; 1.2450x over previous
import dataclasses

import jax
import jax.numpy as jnp
from jax import lax
from jax.experimental import pallas as pl
from jax.experimental.pallas import tpu as pltpu
from jax.experimental.pallas import tpu_sc as plsc

N = 10000
E = 160000
D_IN = 256
HID = 256
HEADS = 4

NC, NS, LANES = 2, 16, 16
NW = NC * NS
NP = 10240
NPS = NP // NS
BN = 1000

KA = 40
KC = 40
KE1 = 80
KE2 = 80

_MESH = plsc.VectorSubcoreMesh(
    core_axis_name="c", subcore_axis_name="s", num_cores=NC, num_subcores=NS)

f32 = jnp.float32
i32 = jnp.int32

_SC_PARAMS = pltpu.CompilerParams()
if "needs_layout_passes" in pltpu.CompilerParams.__dataclass_fields__:
    _SC_PARAMS = dataclasses.replace(_SC_PARAMS, needs_layout_passes=False)


def _fill_zero(buf):
    rows, cols = buf.shape
    z = jnp.zeros((LANES,), f32)
    for r in range(rows):
        for g in range(cols // LANES):
            buf.at[r][pl.ds(g * LANES, LANES)] = z


def _zero_spmem(sp, zbuf, sid):
    zr = zbuf.shape[0]
    reps = NPS // zr

    @pl.loop(0, reps)
    def _(i):
        pltpu.sync_copy(zbuf, sp.at[pl.ds(sid * NPS + i * zr, zr)])


def _hist_body(dst_hbm, out_hbm, idxv, onesv, zbuf, hist_sp):
    cid = lax.axis_index("c")
    sid = lax.axis_index("s")
    w = sid * NC + cid
    ii = lax.iota(i32, LANES)
    row1 = jnp.where(ii == 0, 1.0, 0.0).astype(f32)
    _fill_zero(onesv)
    for r in range(KA):
        onesv.at[r][pl.ds(0, LANES)] = row1
    _fill_zero(zbuf)
    _zero_spmem(hist_sp, zbuf, sid)
    plsc.subcore_barrier()

    @pl.loop(0, (E // KA) // NW)
    def _(j):
        base = (w + j * NW) * KA
        pltpu.sync_copy(dst_hbm.at[pl.ds(base, KA)], idxv)
        pltpu.sync_copy(onesv, hist_sp.at[idxv], add=True)

    plsc.subcore_barrier()
    pltpu.sync_copy(hist_sp.at[pl.ds(sid * NPS, NPS)],
                    out_hbm.at[cid, pl.ds(sid * NPS, NPS)])


def _sc_hist(dst):
    return pl.kernel(
        _hist_body,
        out_type=jax.ShapeDtypeStruct((NC, NP, 128), f32),
        mesh=_MESH,
        scratch_types=[
            pltpu.VMEM((KA,), i32),
            pltpu.VMEM((KA, 128), f32),
            pltpu.VMEM((32, 128), f32),
            pltpu.VMEM_SHARED((NP, 128), f32),
        ],
    )(dst)


def _agg_body(tab_hbm, srcs_hbm, dst_hbm, out_hbm, gidx, sidx, rows, zbuf,
              acc_sp):
    cid = lax.axis_index("c")
    sid = lax.axis_index("s")
    _fill_zero(zbuf)
    _zero_spmem(acc_sp, zbuf, sid)
    plsc.subcore_barrier()

    @pl.loop(0, (E // KC) // NS)
    def _(j):
        base = (sid + j * NS) * KC
        pltpu.sync_copy(srcs_hbm.at[pl.ds(cid * E + base, KC)], gidx)
        pltpu.sync_copy(dst_hbm.at[pl.ds(base, KC)], sidx)
        pltpu.sync_copy(tab_hbm.at[gidx], rows)
        pltpu.sync_copy(rows, acc_sp.at[sidx], add=True)

    plsc.subcore_barrier()
    pltpu.sync_copy(acc_sp.at[pl.ds(sid * NPS, NPS)],
                    out_hbm.at[cid, pl.ds(sid * NPS, NPS)])


def _sc_agg(tab2, src2, dst):
    return pl.kernel(
        _agg_body,
        out_type=jax.ShapeDtypeStruct((NC, NP, 128), f32),
        mesh=_MESH,
        scratch_types=[
            pltpu.VMEM((KC,), i32),
            pltpu.VMEM((KC,), i32),
            pltpu.VMEM((KC, 128), f32),
            pltpu.VMEM((32, 128), f32),
            pltpu.VMEM_SHARED((NP, 128), f32),
        ],
    )(tab2, src2, dst)


def _gat1_body(src_hbm, dst_hbm, asad_hbm, mrep_hbm, ee_hbm, den_hbm,
               mrep_v, srcb, dstb, asb, adb, st80, eeb, zbuf, den_sp):
    cid = lax.axis_index("c")
    sid = lax.axis_index("s")
    pltpu.sync_copy(mrep_hbm, mrep_v)
    _fill_zero(st80)
    _fill_zero(zbuf)
    _zero_spmem(den_sp, zbuf, sid)
    plsc.subcore_barrier()

    io = lax.iota(i32, LANES)

    @pl.loop(0, (E // KE1) // NS)
    def _(j):
        base = (sid + j * NS) * KE1
        pltpu.sync_copy(src_hbm.at[pl.ds(base, KE1)], srcb)
        pltpu.sync_copy(dst_hbm.at[pl.ds(base, KE1)], dstb)
        pltpu.sync_copy(asad_hbm.at[srcb], asb)
        pltpu.sync_copy(asad_hbm.at[dstb], adb)
        for i in range(KE1 // LANES):
            r16 = i * LANES + io
            for hd in range(HEADS):
                hv = jnp.full((LANES,), hd, i32)
                av = plsc.load_gather(asb, [r16, hv])
                bv = plsc.load_gather(adb, [r16, hv + HEADS])
                ev = av + bv
                lr = jnp.maximum(ev, 0.2 * ev)
                fv = jnp.exp(lr - mrep_v.at[hd][pl.ds(0, LANES)])
                plsc.store_scatter(st80, [r16, hv], fv)
                plsc.store_scatter(eeb, [r16 * HEADS + hv], fv)
        pltpu.sync_copy(st80, den_sp.at[dstb], add=True)
        pltpu.sync_copy(eeb, ee_hbm.at[pl.ds(base * HEADS, KE1 * HEADS)])

    plsc.subcore_barrier()
    pltpu.sync_copy(den_sp.at[pl.ds(sid * NPS, NPS)],
                    den_hbm.at[cid, pl.ds(sid * NPS, NPS)])


def _sc_gat1(src, dst, asad128, mrep):
    return pl.kernel(
        _gat1_body,
        out_type=[jax.ShapeDtypeStruct((E * HEADS,), f32),
                  jax.ShapeDtypeStruct((NC, NP, 128), f32)],
        mesh=_MESH,
        scratch_types=[
            pltpu.VMEM((HEADS, 16), f32),
            pltpu.VMEM((KE1,), i32),
            pltpu.VMEM((KE1,), i32),
            pltpu.VMEM((KE1, 128), f32),
            pltpu.VMEM((KE1, 128), f32),
            pltpu.VMEM((KE1, 128), f32),
            pltpu.VMEM((KE1 * HEADS,), f32),
            pltpu.VMEM((32, 128), f32),
            pltpu.VMEM_SHARED((NP, 128), f32),
        ],
        compiler_params=_SC_PARAMS,
    )(src, dst, asad128, mrep)


def _gat2_body(hh2_hbm, srcs_hbm, dst_hbm, ee_hbm, invd_hbm, out_hbm,
               srcb, dstb, eeb, wb, ivb, rows, outst, zbuf, gacc_sp):
    cid = lax.axis_index("c")
    sid = lax.axis_index("s")
    _fill_zero(zbuf)
    _zero_spmem(gacc_sp, zbuf, sid)
    plsc.subcore_barrier()

    io = lax.iota(i32, LANES)
    bf16 = jnp.bfloat16

    @pl.loop(0, (E // KE2) // NS)
    def _(j):
        base = (sid + j * NS) * KE2
        pltpu.sync_copy(srcs_hbm.at[pl.ds(cid * E + base, KE2)], srcb)
        pltpu.sync_copy(dst_hbm.at[pl.ds(base, KE2)], dstb)
        pltpu.sync_copy(ee_hbm.at[pl.ds(base * HEADS, KE2 * HEADS)], eeb)
        pltpu.sync_copy(invd_hbm.at[dstb], ivb)
        pltpu.sync_copy(hh2_hbm.at[srcb], rows)
        for i in range(KE2 // LANES):
            r16 = i * LANES + io
            for hd in range(HEADS):
                hv = jnp.full((LANES,), hd, i32)
                ee = plsc.load_gather(eeb, [r16 * HEADS + hv])
                iv = plsc.load_gather(ivb, [r16, hv])
                plsc.store_scatter(wb, [r16 * HEADS + hv], ee * iv)

        @pl.loop(0, KE2)
        def _(r):
            rv = jnp.zeros((LANES,), i32) + r * HEADS
            w32 = []
            for hd in range(HEADS):
                hv = jnp.full((LANES,), hd, i32)
                wv = plsc.load_gather(wb, [rv + hv])
                w32.append(plsc.pack(wv, wv,
                                     format=plsc.PackFormat.INTERLEAVED))
            for g in range(4):
                acc = jnp.zeros((2 * LANES,), bf16)
                for hd in range(HEADS):
                    off = hd * 64 + g * LANES
                    ri = rows[r, off // 128, pl.ds(off % 128, LANES)]
                    acc = acc + w32[hd] * plsc.bitcast(ri, bf16)
                ev, od = plsc.unpack(acc, format=plsc.PackFormat.INTERLEAVED)
                outst[r, pl.ds(g * 32, LANES)] = ev
                outst[r, pl.ds(g * 32 + LANES, LANES)] = od

        pltpu.sync_copy(outst, gacc_sp.at[dstb], add=True)

    plsc.subcore_barrier()
    pltpu.sync_copy(gacc_sp.at[pl.ds(sid * NPS, NPS)],
                    out_hbm.at[cid, pl.ds(sid * NPS, NPS)])


def _sc_gat2(hh2b, src2, dst, ee4, invd128):
    return pl.kernel(
        _gat2_body,
        out_type=jax.ShapeDtypeStruct((NC, NP, 128), f32),
        mesh=_MESH,
        scratch_types=[
            pltpu.VMEM((KE2,), i32),
            pltpu.VMEM((KE2,), i32),
            pltpu.VMEM((KE2 * HEADS,), f32),
            pltpu.VMEM((KE2 * HEADS,), f32),
            pltpu.VMEM((KE2, 128), f32),
            pltpu.VMEM((KE2, 2, 128), i32),
            pltpu.VMEM((KE2, 128), f32),
            pltpu.VMEM((32, 128), f32),
            pltpu.VMEM_SHARED((NP, 128), f32),
        ],
        compiler_params=_SC_PARAMS,
    )(hh2b, src2, dst, ee4, invd128)


def _mm_body(x_ref, w_ref, o_ref):
    o_ref[...] = jnp.dot(x_ref[...], w_ref[...],
                         preferred_element_type=f32)


def _tc_matmul(x, w):
    m, k = x.shape
    _, n = w.shape
    return pl.pallas_call(
        _mm_body,
        grid=(m // BN,),
        in_specs=[pl.BlockSpec((BN, k), lambda i: (i, 0)),
                  pl.BlockSpec((k, n), lambda i: (0, 0))],
        out_specs=pl.BlockSpec((BN, n), lambda i: (i, 0)),
        out_shape=jax.ShapeDtypeStruct((m, n), f32),
    )(x, w)


def _prescale_body(hx_ref, h0_ref, h1_ref, hp_ref, dinv_ref, degn1_ref):
    degn = h0_ref[...] + h1_ref[...]
    deg = degn + 1.0
    dinv = lax.rsqrt(deg)
    hp_ref[...] = hx_ref[...] * dinv
    dinv_ref[...] = dinv
    degn1_ref[...] = jnp.maximum(degn, 1.0)


def _tc_prescale(hx, h0, h1):
    return pl.pallas_call(
        _prescale_body,
        grid=(N // BN,),
        in_specs=[pl.BlockSpec((BN, HID), lambda i: (i, 0)),
                  pl.BlockSpec((BN, 1), lambda i: (i, 0)),
                  pl.BlockSpec((BN, 1), lambda i: (i, 0))],
        out_specs=[pl.BlockSpec((BN, HID), lambda i: (i, 0)),
                   pl.BlockSpec((BN, 1), lambda i: (i, 0)),
                   pl.BlockSpec((BN, 1), lambda i: (i, 0))],
        out_shape=[jax.ShapeDtypeStruct((N, HID), f32),
                   jax.ShapeDtypeStruct((N, 1), f32),
                   jax.ShapeDtypeStruct((N, 1), f32)],
    )(hx, h0, h1)


def _gcn_gat_body(agg_ref, hp_ref, dinv_ref, bg_ref, wgat_ref, amat_ref,
                  hh_ref, asad_ref, macc_ref):
    i = pl.program_id(0)
    h = jnp.maximum(dinv_ref[...] * (agg_ref[...] + hp_ref[...])
                    + bg_ref[...], 0.0)
    hh = jnp.dot(h, wgat_ref[...], preferred_element_type=f32)
    hh_ref[...] = hh
    asad = jnp.dot(hh, amat_ref[...], preferred_element_type=f32)
    asad_ref[...] = jnp.concatenate(
        [asad, jnp.zeros((asad.shape[0], 120), f32)], axis=1)
    mx = jnp.max(asad, axis=0, keepdims=True)
    mx8 = jnp.broadcast_to(mx, (8, 8))
    prev = jnp.where(i == 0, jnp.full((8, 8), -jnp.inf, f32), macc_ref[...])
    macc_ref[...] = jnp.maximum(prev, mx8)


def _tc_gcn_gat(aggcat, hp, dinv, b_gcn, W_gat, Amat):
    return pl.pallas_call(
        _gcn_gat_body,
        grid=(N // BN,),
        in_specs=[pl.BlockSpec((BN, HID), lambda i: (i, 0)),
                  pl.BlockSpec((BN, HID), lambda i: (i, 0)),
                  pl.BlockSpec((BN, 1), lambda i: (i, 0)),
                  pl.BlockSpec((1, HID), lambda i: (0, 0)),
                  pl.BlockSpec((HID, HEADS * HID), lambda i: (0, 0)),
                  pl.BlockSpec((HEADS * HID, 8), lambda i: (0, 0))],
        out_specs=[pl.BlockSpec((BN, HEADS * HID), lambda i: (i, 0)),
                   pl.BlockSpec((BN, 128), lambda i: (i, 0)),
                   pl.BlockSpec((8, 8), lambda i: (0, 0))],
        out_shape=[jax.ShapeDtypeStruct((N, HEADS * HID), f32),
                   jax.ShapeDtypeStruct((N, 128), f32),
                   jax.ShapeDtypeStruct((8, 8), f32)],
    )(aggcat, hp, dinv, b_gcn, W_gat, Amat)


def _denom_body(den_ref, asad_ref, m4_ref, hh_ref, invd_ref, gself_ref):
    den = 0.5 * (den_ref[0, :, :HEADS] + den_ref[1, :, :HEADS])
    es = asad_ref[:, :HEADS] + asad_ref[:, HEADS:]
    lr = jnp.maximum(es, 0.2 * es)
    se = jnp.exp(lr - m4_ref[...])
    dtot = den + se
    invd = 1.0 / (4.0 * dtot)
    invd_ref[...] = jnp.concatenate(
        [invd, jnp.zeros((invd.shape[0], 124), f32)], axis=1)
    acc = jnp.zeros_like(gself_ref)
    for hd in range(HEADS):
        coef = se[:, hd:hd + 1] * invd[:, hd:hd + 1]
        acc = acc + coef * hh_ref[:, hd * HID:(hd + 1) * HID]
    gself_ref[...] = acc


def _tc_denom(denp, asad, M4r, hh):
    return pl.pallas_call(
        _denom_body,
        grid=(N // BN,),
        in_specs=[pl.BlockSpec((NC, BN, 128), lambda i: (0, i, 0)),
                  pl.BlockSpec((BN, 8), lambda i: (i, 0)),
                  pl.BlockSpec((1, HEADS), lambda i: (0, 0)),
                  pl.BlockSpec((BN, HEADS * HID), lambda i: (i, 0))],
        out_specs=[pl.BlockSpec((BN, 128), lambda i: (i, 0)),
                   pl.BlockSpec((BN, HID), lambda i: (i, 0))],
        out_shape=[jax.ShapeDtypeStruct((N, 128), f32),
                   jax.ShapeDtypeStruct((N, HID), f32)],
    )(denp, asad, M4r, hh)


def _gat_fin_body(graw_ref, gself_ref, bgat_ref, g_ref):
    g_ref[...] = jnp.maximum(graw_ref[...] + gself_ref[...] + bgat_ref[...],
                             0.0)


def _tc_gat_fin(grawcat, gself, b_gat):
    return pl.pallas_call(
        _gat_fin_body,
        grid=(N // BN,),
        in_specs=[pl.BlockSpec((BN, HID), lambda i: (i, 0)),
                  pl.BlockSpec((BN, HID), lambda i: (i, 0)),
                  pl.BlockSpec((1, HID), lambda i: (0, 0))],
        out_specs=pl.BlockSpec((BN, HID), lambda i: (i, 0)),
        out_shape=jax.ShapeDtypeStruct((N, HID), f32),
    )(grawcat, gself, b_gat)


def _head_body(nsum_ref, degn1_ref, g_ref, wl_ref, wr_ref, bs_ref,
               wc1_ref, bc1_ref, wc2_ref, bc2_ref, o_ref):
    neigh = nsum_ref[...] / degn1_ref[...]
    s = (jnp.dot(neigh, wl_ref[...], preferred_element_type=f32)
         + jnp.dot(g_ref[...], wr_ref[...], preferred_element_type=f32)
         + bs_ref[...])
    c = jnp.maximum(jnp.dot(s, wc1_ref[...], preferred_element_type=f32)
                    + bc1_ref[...], 0.0)
    logits = jnp.dot(c, wc2_ref[...], preferred_element_type=f32) + bc2_ref[...]
    o_ref[...] = jax.nn.sigmoid(logits)


def _tc_head(nsumcat, degn1, g, W_sage_l, W_sage_r, bs, W_c1, bc1, W_c2p, bc2):
    return pl.pallas_call(
        _head_body,
        grid=(N // BN,),
        in_specs=[pl.BlockSpec((BN, HID), lambda i: (i, 0)),
                  pl.BlockSpec((BN, 1), lambda i: (i, 0)),
                  pl.BlockSpec((BN, HID), lambda i: (i, 0)),
                  pl.BlockSpec((HID, HID), lambda i: (0, 0)),
                  pl.BlockSpec((HID, HID), lambda i: (0, 0)),
                  pl.BlockSpec((1, HID), lambda i: (0, 0)),
                  pl.BlockSpec((HID, HID // 2), lambda i: (0, 0)),
                  pl.BlockSpec((1, HID // 2), lambda i: (0, 0)),
                  pl.BlockSpec((HID // 2, 8), lambda i: (0, 0)),
                  pl.BlockSpec((1, 8), lambda i: (0, 0))],
        out_specs=pl.BlockSpec((BN, 8), lambda i: (i, 0)),
        out_shape=jax.ShapeDtypeStruct((N, 8), f32),
    )(nsumcat, degn1, g, W_sage_l, W_sage_r, bs, W_c1, bc1, W_c2p, bc2)


def kernel(x, edge_index, W_gcn, b_gcn, W_gat, att_src, att_dst, b_gat,
           W_sage_l, W_sage_r, b_sage, W_c1, b_c1, W_c2, b_c2):
    src = edge_index[0]
    dst = edge_index[1]
    src2 = jnp.concatenate([src, src + N])

    hist = _sc_hist(dst)
    hx = _tc_matmul(x, W_gcn)
    h0 = hist[0, :N, :1]
    h1 = hist[1, :N, :1]
    hp, dinv, degn1 = _tc_prescale(hx, h0, h1)
    hp2 = jnp.concatenate([hp[:, :128], hp[:, 128:]], axis=0)
    agg = _sc_agg(hp2, src2, dst)
    aggcat = jnp.concatenate([agg[0, :N], agg[1, :N]], axis=1)

    eye = jnp.eye(HEADS, dtype=f32)
    As = (att_src[:, :, None] * eye[:, None, :]).reshape(HEADS * HID, HEADS)
    Ad = (att_dst[:, :, None] * eye[:, None, :]).reshape(HEADS * HID, HEADS)
    Amat = jnp.concatenate([As, Ad], axis=1)

    hh, asad, macc = _tc_gcn_gat(aggcat, hp, dinv, b_gcn.reshape(1, HID),
                                 W_gat, Amat)
    ms = macc[0, :HEADS]
    md = macc[0, HEADS:]
    msum = ms + md
    M4 = jnp.maximum(msum, 0.2 * msum)
    mrep = jnp.broadcast_to(M4[:, None], (HEADS, 16))

    ee4, denp = _sc_gat1(src, dst, asad, mrep)
    invd128, gself = _tc_denom(denp, asad[:, :8], M4.reshape(1, HEADS), hh)

    hh4 = hh.reshape(N, HEADS, HID)
    hh2b = jnp.concatenate([hh4[:, :, :128], hh4[:, :, 128:]],
                           axis=0).astype(jnp.bfloat16)
    hh2i = jax.lax.bitcast_convert_type(
        hh2b.reshape(2 * N, 256, 2), i32).reshape(2 * N, 2, 128)
    graw = _sc_gat2(hh2i, src2, dst, ee4, invd128)
    grawcat = jnp.concatenate([graw[0, :N], graw[1, :N]], axis=1)
    perm = [(c // 32) * 32 + ((c % 32) // 2 if c % 2 == 0
                              else 16 + (c % 32) // 2) for c in range(HID)]
    grawcat = grawcat[:, jnp.array(perm, dtype=i32)]
    g = _tc_gat_fin(grawcat, gself, b_gat.reshape(1, HID))

    g2 = jnp.concatenate([g[:, :128], g[:, 128:]], axis=0)
    nsum = _sc_agg(g2, src2, dst)
    nsumcat = jnp.concatenate([nsum[0, :N], nsum[1, :N]], axis=1)

    W_c2p = jnp.concatenate([W_c2, jnp.zeros((HID // 2, 7), f32)], axis=1)
    bc2p = jnp.concatenate([b_c2, jnp.zeros((7,), f32)]).reshape(1, 8)
    out8 = _tc_head(nsumcat, degn1, g, W_sage_l, W_sage_r,
                    b_sage.reshape(1, HID), W_c1, b_c1.reshape(1, HID // 2),
                    W_c2p, bc2p)
    return out8[:, :1]


# speedup vs baseline: 1.4320x; 1.1502x over previous
import dataclasses

import jax
import jax.numpy as jnp
from jax import lax
from jax.experimental import pallas as pl
from jax.experimental.pallas import tpu as pltpu
from jax.experimental.pallas import tpu_sc as plsc

N = 10000
E = 160000
D_IN = 256
HID = 256
HEADS = 4

NC, NS, LANES = 2, 16, 16
NW = NC * NS
NP = 10240
NPS = NP // NS
BN = 1000

KA = 40
KC = 40
KE1 = 80
KE2 = 80

_MESH = plsc.VectorSubcoreMesh(
    core_axis_name="c", subcore_axis_name="s", num_cores=NC, num_subcores=NS)

f32 = jnp.float32
i32 = jnp.int32

_SC_PARAMS = pltpu.CompilerParams()
if "needs_layout_passes" in pltpu.CompilerParams.__dataclass_fields__:
    _SC_PARAMS = dataclasses.replace(_SC_PARAMS, needs_layout_passes=False)


def _fill_zero(buf):
    rows, cols = buf.shape
    z = jnp.zeros((LANES,), f32)
    for r in range(rows):
        for g in range(cols // LANES):
            buf.at[r][pl.ds(g * LANES, LANES)] = z


def _zero_spmem(sp, zbuf, sid):
    zr = zbuf.shape[0]
    reps = NPS // zr

    @pl.loop(0, reps)
    def _(i):
        pltpu.sync_copy(zbuf, sp.at[pl.ds(sid * NPS + i * zr, zr)])


def _hist_body(dst_hbm, out_hbm, idxv, onesv, zbuf, hist_sp):
    cid = lax.axis_index("c")
    sid = lax.axis_index("s")
    w = sid * NC + cid
    ii = lax.iota(i32, LANES)
    row1 = jnp.where(ii == 0, 1.0, 0.0).astype(f32)
    _fill_zero(onesv)
    for r in range(KA):
        onesv.at[r][pl.ds(0, LANES)] = row1
    _fill_zero(zbuf)
    _zero_spmem(hist_sp, zbuf, sid)
    plsc.subcore_barrier()

    @pl.loop(0, (E // KA) // NW)
    def _(j):
        base = (w + j * NW) * KA
        pltpu.sync_copy(dst_hbm.at[pl.ds(base, KA)], idxv)
        pltpu.sync_copy(onesv, hist_sp.at[idxv], add=True)

    plsc.subcore_barrier()
    pltpu.sync_copy(hist_sp.at[pl.ds(sid * NPS, NPS)],
                    out_hbm.at[cid, pl.ds(sid * NPS, NPS)])


def _sc_hist(dst):
    return pl.kernel(
        _hist_body,
        out_type=jax.ShapeDtypeStruct((NC, NP, 128), f32),
        mesh=_MESH,
        scratch_types=[
            pltpu.VMEM((KA,), i32),
            pltpu.VMEM((KA, 128), f32),
            pltpu.VMEM((32, 128), f32),
            pltpu.VMEM_SHARED((NP, 128), f32),
        ],
    )(dst)


def _agg_body(tab_hbm, srcs_hbm, dst_hbm, out_hbm, gidx, sidx, rows, zbuf,
              acc_sp, gsem, ssem):
    cid = lax.axis_index("c")
    sid = lax.axis_index("s")
    _fill_zero(zbuf)
    _zero_spmem(acc_sp, zbuf, sid)
    plsc.subcore_barrier()

    P = 5

    @pl.loop(0, (E // KC) // NS // P)
    def _(jj):
        descs = []
        for p in range(P):
            base = (sid + (jj * P + p) * NS) * KC
            pltpu.sync_copy(srcs_hbm.at[pl.ds(cid * E + base, KC)],
                            gidx.at[p])
            pltpu.sync_copy(dst_hbm.at[pl.ds(base, KC)], sidx.at[p])
            descs.append(pltpu.async_copy(tab_hbm.at[gidx.at[p]],
                                          rows.at[p], gsem))
        sdescs = []
        for p in range(P):
            descs[p].wait()
            sdescs.append(pltpu.async_copy(rows.at[p],
                                           acc_sp.at[sidx.at[p]], ssem,
                                           add=True))
        for p in range(P):
            sdescs[p].wait()

    plsc.subcore_barrier()
    pltpu.sync_copy(acc_sp.at[pl.ds(sid * NPS, NPS)],
                    out_hbm.at[cid, pl.ds(sid * NPS, NPS)])


def _sc_agg(tab2, src2, dst):
    return pl.kernel(
        _agg_body,
        out_type=jax.ShapeDtypeStruct((NC, NP, 128), f32),
        mesh=_MESH,
        scratch_types=[
            pltpu.VMEM((5, KC), i32),
            pltpu.VMEM((5, KC), i32),
            pltpu.VMEM((5, KC, 128), f32),
            pltpu.VMEM((32, 128), f32),
            pltpu.VMEM_SHARED((NP, 128), f32),
            pltpu.SemaphoreType.DMA,
            pltpu.SemaphoreType.DMA,
        ],
    )(tab2, src2, dst)


def _gat1_body(src_hbm, dst_hbm, asad_hbm, mrep_hbm, ee_hbm, den_hbm,
               mrep_v, srcb, dstb, asb, adb, st80, eeb, zbuf, den_sp):
    cid = lax.axis_index("c")
    sid = lax.axis_index("s")
    pltpu.sync_copy(mrep_hbm, mrep_v)
    _fill_zero(st80)
    _fill_zero(zbuf)
    _zero_spmem(den_sp, zbuf, sid)
    plsc.subcore_barrier()

    io = lax.iota(i32, LANES)

    @pl.loop(0, (E // KE1) // NS)
    def _(j):
        base = (sid + j * NS) * KE1
        pltpu.sync_copy(src_hbm.at[pl.ds(base, KE1)], srcb)
        pltpu.sync_copy(dst_hbm.at[pl.ds(base, KE1)], dstb)
        pltpu.sync_copy(asad_hbm.at[srcb], asb)
        pltpu.sync_copy(asad_hbm.at[dstb], adb)
        for i in range(KE1 // LANES):
            r16 = i * LANES + io
            for hd in range(HEADS):
                hv = jnp.full((LANES,), hd, i32)
                av = plsc.load_gather(asb, [r16, hv])
                bv = plsc.load_gather(adb, [r16, hv + HEADS])
                ev = av + bv
                lr = jnp.maximum(ev, 0.2 * ev)
                fv = jnp.exp(lr - mrep_v.at[hd][pl.ds(0, LANES)])
                plsc.store_scatter(st80, [r16, hv], fv)
                plsc.store_scatter(eeb, [r16 * HEADS + hv], fv)
        pltpu.sync_copy(st80, den_sp.at[dstb], add=True)
        pltpu.sync_copy(eeb, ee_hbm.at[pl.ds(base * HEADS, KE1 * HEADS)])

    plsc.subcore_barrier()
    pltpu.sync_copy(den_sp.at[pl.ds(sid * NPS, NPS)],
                    den_hbm.at[cid, pl.ds(sid * NPS, NPS)])


def _sc_gat1(src, dst, asad128, mrep):
    return pl.kernel(
        _gat1_body,
        out_type=[jax.ShapeDtypeStruct((E * HEADS,), f32),
                  jax.ShapeDtypeStruct((NC, NP, 128), f32)],
        mesh=_MESH,
        scratch_types=[
            pltpu.VMEM((HEADS, 16), f32),
            pltpu.VMEM((KE1,), i32),
            pltpu.VMEM((KE1,), i32),
            pltpu.VMEM((KE1, 128), f32),
            pltpu.VMEM((KE1, 128), f32),
            pltpu.VMEM((KE1, 128), f32),
            pltpu.VMEM((KE1 * HEADS,), f32),
            pltpu.VMEM((32, 128), f32),
            pltpu.VMEM_SHARED((NP, 128), f32),
        ],
        compiler_params=_SC_PARAMS,
    )(src, dst, asad128, mrep)


def _gat2_body(hh2_hbm, srcs_hbm, dst_hbm, ee_hbm, invd_hbm, out_hbm,
               srcb, dstb, eeb, wb, ivb, rows, outst, zbuf, gacc_sp):
    cid = lax.axis_index("c")
    sid = lax.axis_index("s")
    _fill_zero(zbuf)
    _zero_spmem(gacc_sp, zbuf, sid)
    plsc.subcore_barrier()

    io = lax.iota(i32, LANES)
    bf16 = jnp.bfloat16

    @pl.loop(0, (E // KE2) // NS)
    def _(j):
        base = (sid + j * NS) * KE2
        pltpu.sync_copy(srcs_hbm.at[pl.ds(cid * E + base, KE2)], srcb)
        pltpu.sync_copy(dst_hbm.at[pl.ds(base, KE2)], dstb)
        pltpu.sync_copy(ee_hbm.at[pl.ds(base * HEADS, KE2 * HEADS)], eeb)
        pltpu.sync_copy(invd_hbm.at[dstb], ivb)
        pltpu.sync_copy(hh2_hbm.at[srcb], rows)
        for i in range(KE2 // LANES):
            r16 = i * LANES + io
            for hd in range(HEADS):
                hv = jnp.full((LANES,), hd, i32)
                ee = plsc.load_gather(eeb, [r16 * HEADS + hv])
                iv = plsc.load_gather(ivb, [r16, hv])
                plsc.store_scatter(wb, [r16 * HEADS + hv], ee * iv)

        @pl.loop(0, KE2)
        def _(r):
            rv = jnp.zeros((LANES,), i32) + r * HEADS
            w32 = []
            for hd in range(HEADS):
                hv = jnp.full((LANES,), hd, i32)
                wv = plsc.load_gather(wb, [rv + hv])
                w32.append(plsc.pack(wv, wv,
                                     format=plsc.PackFormat.INTERLEAVED))
            for g in range(4):
                acc = jnp.zeros((2 * LANES,), bf16)
                for hd in range(HEADS):
                    off = hd * 64 + g * LANES
                    ri = rows[r, off // 128, pl.ds(off % 128, LANES)]
                    acc = acc + w32[hd] * plsc.bitcast(ri, bf16)
                ev, od = plsc.unpack(acc, format=plsc.PackFormat.INTERLEAVED)
                outst[r, pl.ds(g * 32, LANES)] = ev
                outst[r, pl.ds(g * 32 + LANES, LANES)] = od

        pltpu.sync_copy(outst, gacc_sp.at[dstb], add=True)

    plsc.subcore_barrier()
    pltpu.sync_copy(gacc_sp.at[pl.ds(sid * NPS, NPS)],
                    out_hbm.at[cid, pl.ds(sid * NPS, NPS)])


def _sc_gat2(hh2b, src2, dst, ee4, invd128):
    return pl.kernel(
        _gat2_body,
        out_type=jax.ShapeDtypeStruct((NC, NP, 128), f32),
        mesh=_MESH,
        scratch_types=[
            pltpu.VMEM((KE2,), i32),
            pltpu.VMEM((KE2,), i32),
            pltpu.VMEM((KE2 * HEADS,), f32),
            pltpu.VMEM((KE2 * HEADS,), f32),
            pltpu.VMEM((KE2, 128), f32),
            pltpu.VMEM((KE2, 2, 128), i32),
            pltpu.VMEM((KE2, 128), f32),
            pltpu.VMEM((32, 128), f32),
            pltpu.VMEM_SHARED((NP, 128), f32),
        ],
        compiler_params=_SC_PARAMS,
    )(hh2b, src2, dst, ee4, invd128)


def _mm_body(x_ref, w_ref, o_ref):
    o_ref[...] = jnp.dot(x_ref[...], w_ref[...],
                         preferred_element_type=f32)


def _tc_matmul(x, w):
    m, k = x.shape
    _, n = w.shape
    return pl.pallas_call(
        _mm_body,
        grid=(m // BN,),
        in_specs=[pl.BlockSpec((BN, k), lambda i: (i, 0)),
                  pl.BlockSpec((k, n), lambda i: (0, 0))],
        out_specs=pl.BlockSpec((BN, n), lambda i: (i, 0)),
        out_shape=jax.ShapeDtypeStruct((m, n), f32),
    )(x, w)


def _prescale_body(hx_ref, h0_ref, h1_ref, hp_ref, dinv_ref, degn1_ref):
    degn = h0_ref[...] + h1_ref[...]
    deg = degn + 1.0
    dinv = lax.rsqrt(deg)
    hp_ref[...] = hx_ref[...] * dinv
    dinv_ref[...] = dinv
    degn1_ref[...] = jnp.maximum(degn, 1.0)


def _tc_prescale(hx, h0, h1):
    return pl.pallas_call(
        _prescale_body,
        grid=(N // BN,),
        in_specs=[pl.BlockSpec((BN, HID), lambda i: (i, 0)),
                  pl.BlockSpec((BN, 1), lambda i: (i, 0)),
                  pl.BlockSpec((BN, 1), lambda i: (i, 0))],
        out_specs=[pl.BlockSpec((BN, HID), lambda i: (i, 0)),
                   pl.BlockSpec((BN, 1), lambda i: (i, 0)),
                   pl.BlockSpec((BN, 1), lambda i: (i, 0))],
        out_shape=[jax.ShapeDtypeStruct((N, HID), f32),
                   jax.ShapeDtypeStruct((N, 1), f32),
                   jax.ShapeDtypeStruct((N, 1), f32)],
    )(hx, h0, h1)


def _gcn_gat_body(agg_ref, hp_ref, dinv_ref, bg_ref, wgat_ref, amat_ref,
                  hh_ref, asad_ref, macc_ref):
    i = pl.program_id(0)
    h = jnp.maximum(dinv_ref[...] * (agg_ref[...] + hp_ref[...])
                    + bg_ref[...], 0.0)
    hh = jnp.dot(h, wgat_ref[...], preferred_element_type=f32)
    hh_ref[...] = hh
    asad = jnp.dot(hh, amat_ref[...], preferred_element_type=f32)
    asad_ref[...] = jnp.concatenate(
        [asad, jnp.zeros((asad.shape[0], 120), f32)], axis=1)
    mx = jnp.max(asad, axis=0, keepdims=True)
    mx8 = jnp.broadcast_to(mx, (8, 8))
    prev = jnp.where(i == 0, jnp.full((8, 8), -jnp.inf, f32), macc_ref[...])
    macc_ref[...] = jnp.maximum(prev, mx8)


def _tc_gcn_gat(aggcat, hp, dinv, b_gcn, W_gat, Amat):
    return pl.pallas_call(
        _gcn_gat_body,
        grid=(N // BN,),
        in_specs=[pl.BlockSpec((BN, HID), lambda i: (i, 0)),
                  pl.BlockSpec((BN, HID), lambda i: (i, 0)),
                  pl.BlockSpec((BN, 1), lambda i: (i, 0)),
                  pl.BlockSpec((1, HID), lambda i: (0, 0)),
                  pl.BlockSpec((HID, HEADS * HID), lambda i: (0, 0)),
                  pl.BlockSpec((HEADS * HID, 8), lambda i: (0, 0))],
        out_specs=[pl.BlockSpec((BN, HEADS * HID), lambda i: (i, 0)),
                   pl.BlockSpec((BN, 128), lambda i: (i, 0)),
                   pl.BlockSpec((8, 8), lambda i: (0, 0))],
        out_shape=[jax.ShapeDtypeStruct((N, HEADS * HID), f32),
                   jax.ShapeDtypeStruct((N, 128), f32),
                   jax.ShapeDtypeStruct((8, 8), f32)],
    )(aggcat, hp, dinv, b_gcn, W_gat, Amat)


def _denom_body(den_ref, asad_ref, m4_ref, hh_ref, invd_ref, gself_ref):
    den = 0.5 * (den_ref[0, :, :HEADS] + den_ref[1, :, :HEADS])
    es = asad_ref[:, :HEADS] + asad_ref[:, HEADS:]
    lr = jnp.maximum(es, 0.2 * es)
    se = jnp.exp(lr - m4_ref[...])
    dtot = den + se
    invd = 1.0 / (4.0 * dtot)
    invd_ref[...] = jnp.concatenate(
        [invd, jnp.zeros((invd.shape[0], 124), f32)], axis=1)
    acc = jnp.zeros_like(gself_ref)
    for hd in range(HEADS):
        coef = se[:, hd:hd + 1] * invd[:, hd:hd + 1]
        acc = acc + coef * hh_ref[:, hd * HID:(hd + 1) * HID]
    gself_ref[...] = acc


def _tc_denom(denp, asad, M4r, hh):
    return pl.pallas_call(
        _denom_body,
        grid=(N // BN,),
        in_specs=[pl.BlockSpec((NC, BN, 128), lambda i: (0, i, 0)),
                  pl.BlockSpec((BN, 8), lambda i: (i, 0)),
                  pl.BlockSpec((1, HEADS), lambda i: (0, 0)),
                  pl.BlockSpec((BN, HEADS * HID), lambda i: (i, 0))],
        out_specs=[pl.BlockSpec((BN, 128), lambda i: (i, 0)),
                   pl.BlockSpec((BN, HID), lambda i: (i, 0))],
        out_shape=[jax.ShapeDtypeStruct((N, 128), f32),
                   jax.ShapeDtypeStruct((N, HID), f32)],
    )(denp, asad, M4r, hh)


def _gat_fin_body(graw_ref, gself_ref, bgat_ref, g_ref):
    g_ref[...] = jnp.maximum(graw_ref[...] + gself_ref[...] + bgat_ref[...],
                             0.0)


def _tc_gat_fin(grawcat, gself, b_gat):
    return pl.pallas_call(
        _gat_fin_body,
        grid=(N // BN,),
        in_specs=[pl.BlockSpec((BN, HID), lambda i: (i, 0)),
                  pl.BlockSpec((BN, HID), lambda i: (i, 0)),
                  pl.BlockSpec((1, HID), lambda i: (0, 0))],
        out_specs=pl.BlockSpec((BN, HID), lambda i: (i, 0)),
        out_shape=jax.ShapeDtypeStruct((N, HID), f32),
    )(grawcat, gself, b_gat)


def _head_body(nsum_ref, degn1_ref, g_ref, wl_ref, wr_ref, bs_ref,
               wc1_ref, bc1_ref, wc2_ref, bc2_ref, o_ref):
    neigh = nsum_ref[...] / degn1_ref[...]
    s = (jnp.dot(neigh, wl_ref[...], preferred_element_type=f32)
         + jnp.dot(g_ref[...], wr_ref[...], preferred_element_type=f32)
         + bs_ref[...])
    c = jnp.maximum(jnp.dot(s, wc1_ref[...], preferred_element_type=f32)
                    + bc1_ref[...], 0.0)
    logits = jnp.dot(c, wc2_ref[...], preferred_element_type=f32) + bc2_ref[...]
    o_ref[...] = jax.nn.sigmoid(logits)


def _tc_head(nsumcat, degn1, g, W_sage_l, W_sage_r, bs, W_c1, bc1, W_c2p, bc2):
    return pl.pallas_call(
        _head_body,
        grid=(N // BN,),
        in_specs=[pl.BlockSpec((BN, HID), lambda i: (i, 0)),
                  pl.BlockSpec((BN, 1), lambda i: (i, 0)),
                  pl.BlockSpec((BN, HID), lambda i: (i, 0)),
                  pl.BlockSpec((HID, HID), lambda i: (0, 0)),
                  pl.BlockSpec((HID, HID), lambda i: (0, 0)),
                  pl.BlockSpec((1, HID), lambda i: (0, 0)),
                  pl.BlockSpec((HID, HID // 2), lambda i: (0, 0)),
                  pl.BlockSpec((1, HID // 2), lambda i: (0, 0)),
                  pl.BlockSpec((HID // 2, 8), lambda i: (0, 0)),
                  pl.BlockSpec((1, 8), lambda i: (0, 0))],
        out_specs=pl.BlockSpec((BN, 8), lambda i: (i, 0)),
        out_shape=jax.ShapeDtypeStruct((N, 8), f32),
    )(nsumcat, degn1, g, W_sage_l, W_sage_r, bs, W_c1, bc1, W_c2p, bc2)


def kernel(x, edge_index, W_gcn, b_gcn, W_gat, att_src, att_dst, b_gat,
           W_sage_l, W_sage_r, b_sage, W_c1, b_c1, W_c2, b_c2):
    src = edge_index[0]
    dst = edge_index[1]
    src2 = jnp.concatenate([src, src + N])

    hist = _sc_hist(dst)
    hx = _tc_matmul(x, W_gcn)
    h0 = hist[0, :N, :1]
    h1 = hist[1, :N, :1]
    hp, dinv, degn1 = _tc_prescale(hx, h0, h1)
    hp2 = jnp.concatenate([hp[:, :128], hp[:, 128:]], axis=0)
    agg = _sc_agg(hp2, src2, dst)
    aggcat = jnp.concatenate([agg[0, :N], agg[1, :N]], axis=1)

    eye = jnp.eye(HEADS, dtype=f32)
    As = (att_src[:, :, None] * eye[:, None, :]).reshape(HEADS * HID, HEADS)
    Ad = (att_dst[:, :, None] * eye[:, None, :]).reshape(HEADS * HID, HEADS)
    Amat = jnp.concatenate([As, Ad], axis=1)

    hh, asad, macc = _tc_gcn_gat(aggcat, hp, dinv, b_gcn.reshape(1, HID),
                                 W_gat, Amat)
    ms = macc[0, :HEADS]
    md = macc[0, HEADS:]
    msum = ms + md
    M4 = jnp.maximum(msum, 0.2 * msum)
    mrep = jnp.broadcast_to(M4[:, None], (HEADS, 16))

    ee4, denp = _sc_gat1(src, dst, asad, mrep)
    invd128, gself = _tc_denom(denp, asad[:, :8], M4.reshape(1, HEADS), hh)

    hh4 = hh.reshape(N, HEADS, HID)
    hh2b = jnp.concatenate([hh4[:, :, :128], hh4[:, :, 128:]],
                           axis=0).astype(jnp.bfloat16)
    hh2i = jax.lax.bitcast_convert_type(
        hh2b.reshape(2 * N, 256, 2), i32).reshape(2 * N, 2, 128)
    graw = _sc_gat2(hh2i, src2, dst, ee4, invd128)
    grawcat = jnp.concatenate([graw[0, :N], graw[1, :N]], axis=1)
    perm = [(c // 32) * 32 + ((c % 32) // 2 if c % 2 == 0
                              else 16 + (c % 32) // 2) for c in range(HID)]
    grawcat = grawcat[:, jnp.array(perm, dtype=i32)]
    g = _tc_gat_fin(grawcat, gself, b_gat.reshape(1, HID))

    g2 = jnp.concatenate([g[:, :128], g[:, 128:]], axis=0)
    nsum = _sc_agg(g2, src2, dst)
    nsumcat = jnp.concatenate([nsum[0, :N], nsum[1, :N]], axis=1)

    W_c2p = jnp.concatenate([W_c2, jnp.zeros((HID // 2, 7), f32)], axis=1)
    bc2p = jnp.concatenate([b_c2, jnp.zeros((7,), f32)]).reshape(1, 8)
    out8 = _tc_head(nsumcat, degn1, g, W_sage_l, W_sage_r,
                    b_sage.reshape(1, HID), W_c1, b_c1.reshape(1, HID // 2),
                    W_c2p, bc2p)
    return out8[:, :1]


# speedup vs baseline: 1.5771x; 1.1013x over previous
import dataclasses

import jax
import jax.numpy as jnp
from jax import lax
from jax.experimental import pallas as pl
from jax.experimental.pallas import tpu as pltpu
from jax.experimental.pallas import tpu_sc as plsc

N = 10000
E = 160000
D_IN = 256
HID = 256
HEADS = 4

NC, NS, LANES = 2, 16, 16
NW = NC * NS
NP = 10240
NPS = NP // NS
BN = 1000

KA = 40
KC = 40
KE1 = 80
KE2 = 80

_MESH = plsc.VectorSubcoreMesh(
    core_axis_name="c", subcore_axis_name="s", num_cores=NC, num_subcores=NS)

f32 = jnp.float32
i32 = jnp.int32

_SC_PARAMS = pltpu.CompilerParams()
if "needs_layout_passes" in pltpu.CompilerParams.__dataclass_fields__:
    _SC_PARAMS = dataclasses.replace(_SC_PARAMS, needs_layout_passes=False)


def _fill_zero(buf):
    rows, cols = buf.shape
    z = jnp.zeros((LANES,), f32)
    for r in range(rows):
        for g in range(cols // LANES):
            buf.at[r][pl.ds(g * LANES, LANES)] = z


def _zero_spmem(sp, zbuf, sid):
    zr = zbuf.shape[0]
    reps = NPS // zr

    @pl.loop(0, reps)
    def _(i):
        pltpu.sync_copy(zbuf, sp.at[pl.ds(sid * NPS + i * zr, zr)])


def _hist_body(dst_hbm, out_hbm, idxv, onesv, zbuf, hist_sp):
    cid = lax.axis_index("c")
    sid = lax.axis_index("s")
    w = sid * NC + cid
    ii = lax.iota(i32, LANES)
    row1 = jnp.where(ii == 0, 1.0, 0.0).astype(f32)
    _fill_zero(onesv)
    for r in range(KA):
        onesv.at[r][pl.ds(0, LANES)] = row1
    _fill_zero(zbuf)
    _zero_spmem(hist_sp, zbuf, sid)
    plsc.subcore_barrier()

    @pl.loop(0, (E // KA) // NW)
    def _(j):
        base = (w + j * NW) * KA
        pltpu.sync_copy(dst_hbm.at[pl.ds(base, KA)], idxv)
        pltpu.sync_copy(onesv, hist_sp.at[idxv], add=True)

    plsc.subcore_barrier()
    pltpu.sync_copy(hist_sp.at[pl.ds(sid * NPS, NPS)],
                    out_hbm.at[cid, pl.ds(sid * NPS, NPS)])


def _sc_hist(dst):
    return pl.kernel(
        _hist_body,
        out_type=jax.ShapeDtypeStruct((NC, NP, 128), f32),
        mesh=_MESH,
        scratch_types=[
            pltpu.VMEM((KA,), i32),
            pltpu.VMEM((KA, 128), f32),
            pltpu.VMEM((32, 128), f32),
            pltpu.VMEM_SHARED((NP, 128), f32),
        ],
    )(dst)


def _agg_body(tab_hbm, srcs_hbm, dst_hbm, out_hbm, gidx, sidx, rows, zbuf,
              acc_sp, gsem, ssem):
    cid = lax.axis_index("c")
    sid = lax.axis_index("s")
    _fill_zero(zbuf)
    _zero_spmem(acc_sp, zbuf, sid)
    plsc.subcore_barrier()

    P = 5

    @pl.loop(0, (E // KC) // NS // P)
    def _(jj):
        descs = []
        for p in range(P):
            base = (sid + (jj * P + p) * NS) * KC
            pltpu.sync_copy(srcs_hbm.at[pl.ds(cid * E + base, KC)],
                            gidx.at[p])
            pltpu.sync_copy(dst_hbm.at[pl.ds(base, KC)], sidx.at[p])
            descs.append(pltpu.async_copy(tab_hbm.at[gidx.at[p]],
                                          rows.at[p], gsem))
        sdescs = []
        for p in range(P):
            descs[p].wait()
            sdescs.append(pltpu.async_copy(rows.at[p],
                                           acc_sp.at[sidx.at[p]], ssem,
                                           add=True))
        for p in range(P):
            sdescs[p].wait()

    plsc.subcore_barrier()
    pltpu.sync_copy(acc_sp.at[pl.ds(sid * NPS, NPS)],
                    out_hbm.at[cid, pl.ds(sid * NPS, NPS)])


def _sc_agg(tab2, src2, dst):
    return pl.kernel(
        _agg_body,
        out_type=jax.ShapeDtypeStruct((NC, NP, 128), f32),
        mesh=_MESH,
        scratch_types=[
            pltpu.VMEM((5, KC), i32),
            pltpu.VMEM((5, KC), i32),
            pltpu.VMEM((5, KC, 128), f32),
            pltpu.VMEM((32, 128), f32),
            pltpu.VMEM_SHARED((NP, 128), f32),
            pltpu.SemaphoreType.DMA,
            pltpu.SemaphoreType.DMA,
        ],
    )(tab2, src2, dst)


def _gat1_body(src_hbm, dst_hbm, asad_hbm, mrep_hbm, ee_hbm, den_hbm,
               mrep_v, srcb, dstb, asb, adb, st80, eeb, zbuf, den_sp):
    cid = lax.axis_index("c")
    sid = lax.axis_index("s")
    pltpu.sync_copy(mrep_hbm, mrep_v)
    _fill_zero(st80)
    _fill_zero(zbuf)
    _zero_spmem(den_sp, zbuf, sid)
    plsc.subcore_barrier()

    io = lax.iota(i32, LANES)
    w = sid * NC + cid
    nch = E // KE1

    @pl.loop(0, (nch + NW - 1) // NW)
    def _(j):
        c = w + j * NW

        @pl.when(c < nch)
        def _():
            base = c * KE1
            pltpu.sync_copy(src_hbm.at[pl.ds(base, KE1)], srcb)
            pltpu.sync_copy(dst_hbm.at[pl.ds(base, KE1)], dstb)
            pltpu.sync_copy(asad_hbm.at[srcb], asb)
            pltpu.sync_copy(asad_hbm.at[dstb], adb)
            for i in range(KE1 // LANES):
                r16 = i * LANES + io
                for hd in range(HEADS):
                    hv = jnp.full((LANES,), hd, i32)
                    av = plsc.load_gather(asb, [r16, hv])
                    bv = plsc.load_gather(adb, [r16, hv + HEADS])
                    ev = av + bv
                    lr = jnp.maximum(ev, 0.2 * ev)
                    fv = jnp.exp(lr - mrep_v.at[hd][pl.ds(0, LANES)])
                    plsc.store_scatter(st80, [r16, hv], fv)
                    plsc.store_scatter(eeb, [r16 * HEADS + hv], fv)
            pltpu.sync_copy(st80, den_sp.at[dstb], add=True)
            pltpu.sync_copy(eeb, ee_hbm.at[pl.ds(base * HEADS, KE1 * HEADS)])

    plsc.subcore_barrier()
    pltpu.sync_copy(den_sp.at[pl.ds(sid * NPS, NPS)],
                    den_hbm.at[cid, pl.ds(sid * NPS, NPS)])


def _sc_gat1(src, dst, asad128, mrep):
    return pl.kernel(
        _gat1_body,
        out_type=[jax.ShapeDtypeStruct((E * HEADS,), f32),
                  jax.ShapeDtypeStruct((NC, NP, 128), f32)],
        mesh=_MESH,
        scratch_types=[
            pltpu.VMEM((HEADS, 16), f32),
            pltpu.VMEM((KE1,), i32),
            pltpu.VMEM((KE1,), i32),
            pltpu.VMEM((KE1, 128), f32),
            pltpu.VMEM((KE1, 128), f32),
            pltpu.VMEM((KE1, 128), f32),
            pltpu.VMEM((KE1 * HEADS,), f32),
            pltpu.VMEM((32, 128), f32),
            pltpu.VMEM_SHARED((NP, 128), f32),
        ],
        compiler_params=_SC_PARAMS,
    )(src, dst, asad128, mrep)


def _gat2_body(hh2_hbm, srcs_hbm, dst_hbm, ee_hbm, invd_hbm, out_hbm,
               srcb, dstb, eeb, wb, ivb, rows, outst, zbuf, gacc_sp):
    cid = lax.axis_index("c")
    sid = lax.axis_index("s")
    _fill_zero(zbuf)
    _zero_spmem(gacc_sp, zbuf, sid)
    plsc.subcore_barrier()

    io = lax.iota(i32, LANES)
    bf16 = jnp.bfloat16

    @pl.loop(0, (E // KE2) // NS)
    def _(j):
        base = (sid + j * NS) * KE2
        pltpu.sync_copy(srcs_hbm.at[pl.ds(cid * E + base, KE2)], srcb)
        pltpu.sync_copy(dst_hbm.at[pl.ds(base, KE2)], dstb)
        pltpu.sync_copy(ee_hbm.at[pl.ds(base * HEADS, KE2 * HEADS)], eeb)
        pltpu.sync_copy(invd_hbm.at[dstb], ivb)
        pltpu.sync_copy(hh2_hbm.at[srcb], rows)
        for i in range(KE2 // LANES):
            r16 = i * LANES + io
            for hd in range(HEADS):
                hv = jnp.full((LANES,), hd, i32)
                ee = plsc.load_gather(eeb, [r16 * HEADS + hv])
                iv = plsc.load_gather(ivb, [r16, hv])
                plsc.store_scatter(wb, [r16 * HEADS + hv], ee * iv)

        @pl.loop(0, KE2)
        def _(r):
            rv = jnp.zeros((LANES,), i32) + r * HEADS
            w32 = []
            for hd in range(HEADS):
                hv = jnp.full((LANES,), hd, i32)
                wv = plsc.load_gather(wb, [rv + hv])
                w32.append(plsc.pack(wv, wv,
                                     format=plsc.PackFormat.INTERLEAVED))
            for g in range(4):
                acc = jnp.zeros((2 * LANES,), bf16)
                for hd in range(HEADS):
                    off = hd * 64 + g * LANES
                    ri = rows[r, off // 128, pl.ds(off % 128, LANES)]
                    acc = acc + w32[hd] * plsc.bitcast(ri, bf16)
                ev, od = plsc.unpack(acc, format=plsc.PackFormat.INTERLEAVED)
                outst[r, pl.ds(g * 32, LANES)] = ev
                outst[r, pl.ds(g * 32 + LANES, LANES)] = od

        pltpu.sync_copy(outst, gacc_sp.at[dstb], add=True)

    plsc.subcore_barrier()
    pltpu.sync_copy(gacc_sp.at[pl.ds(sid * NPS, NPS)],
                    out_hbm.at[cid, pl.ds(sid * NPS, NPS)])


def _sc_gat2(hh2b, src2, dst, ee4, invd128):
    return pl.kernel(
        _gat2_body,
        out_type=jax.ShapeDtypeStruct((NC, NP, 128), f32),
        mesh=_MESH,
        scratch_types=[
            pltpu.VMEM((KE2,), i32),
            pltpu.VMEM((KE2,), i32),
            pltpu.VMEM((KE2 * HEADS,), f32),
            pltpu.VMEM((KE2 * HEADS,), f32),
            pltpu.VMEM((KE2, 128), f32),
            pltpu.VMEM((KE2, 2, 128), i32),
            pltpu.VMEM((KE2, 128), f32),
            pltpu.VMEM((32, 128), f32),
            pltpu.VMEM_SHARED((NP, 128), f32),
        ],
        compiler_params=_SC_PARAMS,
    )(hh2b, src2, dst, ee4, invd128)


def _mm_body(x_ref, w_ref, o_ref):
    o_ref[...] = jnp.dot(x_ref[...], w_ref[...],
                         preferred_element_type=f32)


def _tc_matmul(x, w):
    m, k = x.shape
    _, n = w.shape
    return pl.pallas_call(
        _mm_body,
        grid=(m // BN,),
        in_specs=[pl.BlockSpec((BN, k), lambda i: (i, 0)),
                  pl.BlockSpec((k, n), lambda i: (0, 0))],
        out_specs=pl.BlockSpec((BN, n), lambda i: (i, 0)),
        out_shape=jax.ShapeDtypeStruct((m, n), f32),
    )(x, w)


def _prescale_body(hx_ref, h0_ref, h1_ref, hp_ref, dinv_ref, degn1_ref):
    degn = h0_ref[...] + h1_ref[...]
    deg = degn + 1.0
    dinv = lax.rsqrt(deg)
    hp_ref[...] = hx_ref[...] * dinv
    dinv_ref[...] = dinv
    degn1_ref[...] = jnp.maximum(degn, 1.0)


def _tc_prescale(hx, h0, h1):
    return pl.pallas_call(
        _prescale_body,
        grid=(N // BN,),
        in_specs=[pl.BlockSpec((BN, HID), lambda i: (i, 0)),
                  pl.BlockSpec((BN, 1), lambda i: (i, 0)),
                  pl.BlockSpec((BN, 1), lambda i: (i, 0))],
        out_specs=[pl.BlockSpec((BN, HID), lambda i: (i, 0)),
                   pl.BlockSpec((BN, 1), lambda i: (i, 0)),
                   pl.BlockSpec((BN, 1), lambda i: (i, 0))],
        out_shape=[jax.ShapeDtypeStruct((N, HID), f32),
                   jax.ShapeDtypeStruct((N, 1), f32),
                   jax.ShapeDtypeStruct((N, 1), f32)],
    )(hx, h0, h1)


def _gcn_gat_body(agg_ref, hp_ref, dinv_ref, bg_ref, wgat_ref, amat_ref,
                  hh_ref, asad_ref, macc_ref):
    i = pl.program_id(0)
    h = jnp.maximum(dinv_ref[...] * (agg_ref[...] + hp_ref[...])
                    + bg_ref[...], 0.0)
    hh = jnp.dot(h, wgat_ref[...], preferred_element_type=f32)
    hh_ref[...] = hh
    asad = jnp.dot(hh, amat_ref[...], preferred_element_type=f32)
    asad_ref[...] = jnp.concatenate(
        [asad, jnp.zeros((asad.shape[0], 120), f32)], axis=1)
    mx = jnp.max(asad, axis=0, keepdims=True)
    mx8 = jnp.broadcast_to(mx, (8, 8))
    prev = jnp.where(i == 0, jnp.full((8, 8), -jnp.inf, f32), macc_ref[...])
    macc_ref[...] = jnp.maximum(prev, mx8)


def _tc_gcn_gat(aggcat, hp, dinv, b_gcn, W_gat, Amat):
    return pl.pallas_call(
        _gcn_gat_body,
        grid=(N // BN,),
        in_specs=[pl.BlockSpec((BN, HID), lambda i: (i, 0)),
                  pl.BlockSpec((BN, HID), lambda i: (i, 0)),
                  pl.BlockSpec((BN, 1), lambda i: (i, 0)),
                  pl.BlockSpec((1, HID), lambda i: (0, 0)),
                  pl.BlockSpec((HID, HEADS * HID), lambda i: (0, 0)),
                  pl.BlockSpec((HEADS * HID, 8), lambda i: (0, 0))],
        out_specs=[pl.BlockSpec((BN, HEADS * HID), lambda i: (i, 0)),
                   pl.BlockSpec((BN, 128), lambda i: (i, 0)),
                   pl.BlockSpec((8, 8), lambda i: (0, 0))],
        out_shape=[jax.ShapeDtypeStruct((N, HEADS * HID), f32),
                   jax.ShapeDtypeStruct((N, 128), f32),
                   jax.ShapeDtypeStruct((8, 8), f32)],
    )(aggcat, hp, dinv, b_gcn, W_gat, Amat)


def _denom_body(den_ref, asad_ref, m4_ref, hh_ref, invd_ref, gself_ref):
    den = den_ref[0, :, :HEADS] + den_ref[1, :, :HEADS]
    es = asad_ref[:, :HEADS] + asad_ref[:, HEADS:]
    lr = jnp.maximum(es, 0.2 * es)
    se = jnp.exp(lr - m4_ref[...])
    dtot = den + se
    invd = 1.0 / (4.0 * dtot)
    invd_ref[...] = jnp.concatenate(
        [invd, jnp.zeros((invd.shape[0], 124), f32)], axis=1)
    acc = jnp.zeros_like(gself_ref)
    for hd in range(HEADS):
        coef = se[:, hd:hd + 1] * invd[:, hd:hd + 1]
        acc = acc + coef * hh_ref[:, hd * HID:(hd + 1) * HID]
    gself_ref[...] = acc


def _tc_denom(denp, asad, M4r, hh):
    return pl.pallas_call(
        _denom_body,
        grid=(N // BN,),
        in_specs=[pl.BlockSpec((NC, BN, 128), lambda i: (0, i, 0)),
                  pl.BlockSpec((BN, 8), lambda i: (i, 0)),
                  pl.BlockSpec((1, HEADS), lambda i: (0, 0)),
                  pl.BlockSpec((BN, HEADS * HID), lambda i: (i, 0))],
        out_specs=[pl.BlockSpec((BN, 128), lambda i: (i, 0)),
                   pl.BlockSpec((BN, HID), lambda i: (i, 0))],
        out_shape=[jax.ShapeDtypeStruct((N, 128), f32),
                   jax.ShapeDtypeStruct((N, HID), f32)],
    )(denp, asad, M4r, hh)


def _gat_fin_body(graw_ref, gself_ref, bgat_ref, g_ref):
    g_ref[...] = jnp.maximum(graw_ref[...] + gself_ref[...] + bgat_ref[...],
                             0.0)


def _tc_gat_fin(grawcat, gself, b_gat):
    return pl.pallas_call(
        _gat_fin_body,
        grid=(N // BN,),
        in_specs=[pl.BlockSpec((BN, HID), lambda i: (i, 0)),
                  pl.BlockSpec((BN, HID), lambda i: (i, 0)),
                  pl.BlockSpec((1, HID), lambda i: (0, 0))],
        out_specs=pl.BlockSpec((BN, HID), lambda i: (i, 0)),
        out_shape=jax.ShapeDtypeStruct((N, HID), f32),
    )(grawcat, gself, b_gat)


def _head_body(nsum_ref, degn1_ref, g_ref, wl_ref, wr_ref, bs_ref,
               wc1_ref, bc1_ref, wc2_ref, bc2_ref, o_ref):
    neigh = nsum_ref[...] / degn1_ref[...]
    s = (jnp.dot(neigh, wl_ref[...], preferred_element_type=f32)
         + jnp.dot(g_ref[...], wr_ref[...], preferred_element_type=f32)
         + bs_ref[...])
    c = jnp.maximum(jnp.dot(s, wc1_ref[...], preferred_element_type=f32)
                    + bc1_ref[...], 0.0)
    logits = jnp.dot(c, wc2_ref[...], preferred_element_type=f32) + bc2_ref[...]
    o_ref[...] = jax.nn.sigmoid(logits)


def _tc_head(nsumcat, degn1, g, W_sage_l, W_sage_r, bs, W_c1, bc1, W_c2p, bc2):
    return pl.pallas_call(
        _head_body,
        grid=(N // BN,),
        in_specs=[pl.BlockSpec((BN, HID), lambda i: (i, 0)),
                  pl.BlockSpec((BN, 1), lambda i: (i, 0)),
                  pl.BlockSpec((BN, HID), lambda i: (i, 0)),
                  pl.BlockSpec((HID, HID), lambda i: (0, 0)),
                  pl.BlockSpec((HID, HID), lambda i: (0, 0)),
                  pl.BlockSpec((1, HID), lambda i: (0, 0)),
                  pl.BlockSpec((HID, HID // 2), lambda i: (0, 0)),
                  pl.BlockSpec((1, HID // 2), lambda i: (0, 0)),
                  pl.BlockSpec((HID // 2, 8), lambda i: (0, 0)),
                  pl.BlockSpec((1, 8), lambda i: (0, 0))],
        out_specs=pl.BlockSpec((BN, 8), lambda i: (i, 0)),
        out_shape=jax.ShapeDtypeStruct((N, 8), f32),
    )(nsumcat, degn1, g, W_sage_l, W_sage_r, bs, W_c1, bc1, W_c2p, bc2)


def kernel(x, edge_index, W_gcn, b_gcn, W_gat, att_src, att_dst, b_gat,
           W_sage_l, W_sage_r, b_sage, W_c1, b_c1, W_c2, b_c2):
    src = edge_index[0]
    dst = edge_index[1]
    src2 = jnp.concatenate([src, src + N])

    hist = _sc_hist(dst)
    hx = _tc_matmul(x, W_gcn)
    h0 = hist[0, :N, :1]
    h1 = hist[1, :N, :1]
    hp, dinv, degn1 = _tc_prescale(hx, h0, h1)
    hp2 = jnp.concatenate([hp[:, :128], hp[:, 128:]], axis=0)
    agg = _sc_agg(hp2, src2, dst)
    aggcat = jnp.concatenate([agg[0, :N], agg[1, :N]], axis=1)

    eye = jnp.eye(HEADS, dtype=f32)
    As = (att_src[:, :, None] * eye[:, None, :]).reshape(HEADS * HID, HEADS)
    Ad = (att_dst[:, :, None] * eye[:, None, :]).reshape(HEADS * HID, HEADS)
    Amat = jnp.concatenate([As, Ad], axis=1)

    hh, asad, macc = _tc_gcn_gat(aggcat, hp, dinv, b_gcn.reshape(1, HID),
                                 W_gat, Amat)
    ms = macc[0, :HEADS]
    md = macc[0, HEADS:]
    msum = ms + md
    M4 = jnp.maximum(msum, 0.2 * msum)
    mrep = jnp.broadcast_to(M4[:, None], (HEADS, 16))

    ee4, denp = _sc_gat1(src, dst, asad, mrep)
    invd128, gself = _tc_denom(denp, asad[:, :8], M4.reshape(1, HEADS), hh)

    hh4 = hh.reshape(N, HEADS, HID)
    hh2b = jnp.concatenate([hh4[:, :, :128], hh4[:, :, 128:]],
                           axis=0).astype(jnp.bfloat16)
    hh2i = jax.lax.bitcast_convert_type(
        hh2b.reshape(2 * N, 256, 2), i32).reshape(2 * N, 2, 128)
    graw = _sc_gat2(hh2i, src2, dst, ee4, invd128)
    grawcat = jnp.concatenate([graw[0, :N], graw[1, :N]], axis=1)
    perm = [(c // 32) * 32 + ((c % 32) // 2 if c % 2 == 0
                              else 16 + (c % 32) // 2) for c in range(HID)]
    grawcat = grawcat[:, jnp.array(perm, dtype=i32)]
    g = _tc_gat_fin(grawcat, gself, b_gat.reshape(1, HID))

    g2 = jnp.concatenate([g[:, :128], g[:, 128:]], axis=0)
    nsum = _sc_agg(g2, src2, dst)
    nsumcat = jnp.concatenate([nsum[0, :N], nsum[1, :N]], axis=1)

    W_c2p = jnp.concatenate([W_c2, jnp.zeros((HID // 2, 7), f32)], axis=1)
    bc2p = jnp.concatenate([b_c2, jnp.zeros((7,), f32)]).reshape(1, 8)
    out8 = _tc_head(nsumcat, degn1, g, W_sage_l, W_sage_r,
                    b_sage.reshape(1, HID), W_c1, b_c1.reshape(1, HID // 2),
                    W_c2p, bc2p)
    return out8[:, :1]


# speedup vs baseline: 1.7225x; 1.0921x over previous
import dataclasses

import jax
import jax.numpy as jnp
from jax import lax
from jax.experimental import pallas as pl
from jax.experimental.pallas import tpu as pltpu
from jax.experimental.pallas import tpu_sc as plsc

N = 10000
E = 160000
D_IN = 256
HID = 256
HEADS = 4

NC, NS, LANES = 2, 16, 16
NW = NC * NS
NP = 10240
NPS = NP // NS
BN = 1000

KA = 40
KC = 40
KE1 = 80
KE2 = 80

_MESH = plsc.VectorSubcoreMesh(
    core_axis_name="c", subcore_axis_name="s", num_cores=NC, num_subcores=NS)

f32 = jnp.float32
i32 = jnp.int32

_SC_PARAMS = pltpu.CompilerParams()
if "needs_layout_passes" in pltpu.CompilerParams.__dataclass_fields__:
    _SC_PARAMS = dataclasses.replace(_SC_PARAMS, needs_layout_passes=False)


def _fill_zero(buf):
    rows, cols = buf.shape
    z = jnp.zeros((LANES,), f32)
    for r in range(rows):
        for g in range(cols // LANES):
            buf.at[r][pl.ds(g * LANES, LANES)] = z


def _zero_spmem(sp, zbuf, sid):
    zr = zbuf.shape[0]
    reps = NPS // zr

    @pl.loop(0, reps)
    def _(i):
        pltpu.sync_copy(zbuf, sp.at[pl.ds(sid * NPS + i * zr, zr)])


def _hist_body(dst_hbm, out_hbm, idxv, onesv, zbuf, hist_sp):
    cid = lax.axis_index("c")
    sid = lax.axis_index("s")
    w = sid * NC + cid
    ii = lax.iota(i32, LANES)
    row1 = jnp.where(ii == 0, 1.0, 0.0).astype(f32)
    _fill_zero(onesv)
    for r in range(KA):
        onesv.at[r][pl.ds(0, LANES)] = row1
    _fill_zero(zbuf)
    _zero_spmem(hist_sp, zbuf, sid)
    plsc.subcore_barrier()

    @pl.loop(0, (E // KA) // NW)
    def _(j):
        base = (w + j * NW) * KA
        pltpu.sync_copy(dst_hbm.at[pl.ds(base, KA)], idxv)
        pltpu.sync_copy(onesv, hist_sp.at[idxv], add=True)

    plsc.subcore_barrier()
    pltpu.sync_copy(hist_sp.at[pl.ds(sid * NPS, NPS)],
                    out_hbm.at[cid, pl.ds(sid * NPS, NPS)])


def _sc_hist(dst):
    return pl.kernel(
        _hist_body,
        out_type=jax.ShapeDtypeStruct((NC, NP, 128), f32),
        mesh=_MESH,
        scratch_types=[
            pltpu.VMEM((KA,), i32),
            pltpu.VMEM((KA, 128), f32),
            pltpu.VMEM((32, 128), f32),
            pltpu.VMEM_SHARED((NP, 128), f32),
        ],
    )(dst)


def _agg_body(tab_hbm, srcs_hbm, dst_hbm, out_hbm, gidx, sidx, rows, zbuf,
              acc_sp, gsem, ssem):
    cid = lax.axis_index("c")
    sid = lax.axis_index("s")
    _fill_zero(zbuf)
    _zero_spmem(acc_sp, zbuf, sid)
    plsc.subcore_barrier()

    P = 5

    @pl.loop(0, (E // KC) // NS // P)
    def _(jj):
        descs = []
        for p in range(P):
            base = (sid + (jj * P + p) * NS) * KC
            pltpu.sync_copy(srcs_hbm.at[pl.ds(cid * E + base, KC)],
                            gidx.at[p])
            pltpu.sync_copy(dst_hbm.at[pl.ds(base, KC)], sidx.at[p])
            descs.append(pltpu.async_copy(tab_hbm.at[gidx.at[p]],
                                          rows.at[p], gsem.at[p]))
        sdescs = []
        for p in range(P):
            descs[p].wait()
            sdescs.append(pltpu.async_copy(rows.at[p],
                                           acc_sp.at[sidx.at[p]], ssem.at[p],
                                           add=True))
        for p in range(P):
            sdescs[p].wait()

    plsc.subcore_barrier()
    pltpu.sync_copy(acc_sp.at[pl.ds(sid * NPS, NPS)],
                    out_hbm.at[cid, pl.ds(sid * NPS, NPS)])


def _sc_agg(tab2, src2, dst):
    return pl.kernel(
        _agg_body,
        out_type=jax.ShapeDtypeStruct((NC, NP, 128), f32),
        mesh=_MESH,
        scratch_types=[
            pltpu.VMEM((5, KC), i32),
            pltpu.VMEM((5, KC), i32),
            pltpu.VMEM((5, KC, 128), f32),
            pltpu.VMEM((32, 128), f32),
            pltpu.VMEM_SHARED((NP, 128), f32),
            pltpu.SemaphoreType.DMA((5,)),
            pltpu.SemaphoreType.DMA((5,)),
        ],
    )(tab2, src2, dst)


def _gat1_body(src_hbm, dst_hbm, asad_hbm, mrep_hbm, ee_hbm, den_hbm,
               mrep_v, srcb, dstb, asb, adb, st80, eeb, zbuf, den_sp):
    cid = lax.axis_index("c")
    sid = lax.axis_index("s")
    pltpu.sync_copy(mrep_hbm, mrep_v)
    _fill_zero(st80)
    _fill_zero(zbuf)
    _zero_spmem(den_sp, zbuf, sid)
    plsc.subcore_barrier()

    io = lax.iota(i32, LANES)
    w = sid * NC + cid
    nch = E // KE1

    @pl.loop(0, (nch + NW - 1) // NW)
    def _(j):
        c = w + j * NW

        @pl.when(c < nch)
        def _():
            base = c * KE1
            pltpu.sync_copy(src_hbm.at[pl.ds(base, KE1)], srcb)
            pltpu.sync_copy(dst_hbm.at[pl.ds(base, KE1)], dstb)
            pltpu.sync_copy(asad_hbm.at[srcb], asb)
            pltpu.sync_copy(asad_hbm.at[dstb], adb)
            for i in range(KE1 // LANES):
                r16 = i * LANES + io
                for hd in range(HEADS):
                    hv = jnp.full((LANES,), hd, i32)
                    av = plsc.load_gather(asb, [r16, hv])
                    bv = plsc.load_gather(adb, [r16, hv + HEADS])
                    ev = av + bv
                    lr = jnp.maximum(ev, 0.2 * ev)
                    fv = jnp.exp(lr - mrep_v.at[hd][pl.ds(0, LANES)])
                    plsc.store_scatter(st80, [r16, hv], fv)
                    plsc.store_scatter(eeb, [r16 * HEADS + hv], fv)
            pltpu.sync_copy(st80, den_sp.at[dstb], add=True)
            pltpu.sync_copy(eeb, ee_hbm.at[pl.ds(base * HEADS, KE1 * HEADS)])

    plsc.subcore_barrier()
    pltpu.sync_copy(den_sp.at[pl.ds(sid * NPS, NPS)],
                    den_hbm.at[cid, pl.ds(sid * NPS, NPS)])


def _sc_gat1(src, dst, asad128, mrep):
    return pl.kernel(
        _gat1_body,
        out_type=[jax.ShapeDtypeStruct((E * HEADS,), f32),
                  jax.ShapeDtypeStruct((NC, NP, 128), f32)],
        mesh=_MESH,
        scratch_types=[
            pltpu.VMEM((HEADS, 16), f32),
            pltpu.VMEM((KE1,), i32),
            pltpu.VMEM((KE1,), i32),
            pltpu.VMEM((KE1, 128), f32),
            pltpu.VMEM((KE1, 128), f32),
            pltpu.VMEM((KE1, 128), f32),
            pltpu.VMEM((KE1 * HEADS,), f32),
            pltpu.VMEM((32, 128), f32),
            pltpu.VMEM_SHARED((NP, 128), f32),
        ],
        compiler_params=_SC_PARAMS,
    )(src, dst, asad128, mrep)


def _gat2_body(hh2_hbm, srcs_hbm, dst_hbm, ee_hbm, invd_hbm, out_hbm,
               srcb, dstb, eeb, wb, ivb, rows, outst, zbuf, gacc_sp,
               isem, gsem, vsem):
    cid = lax.axis_index("c")
    sid = lax.axis_index("s")
    _fill_zero(zbuf)
    _zero_spmem(gacc_sp, zbuf, sid)
    plsc.subcore_barrier()

    io = lax.iota(i32, LANES)
    bf16 = jnp.bfloat16

    @pl.loop(0, (E // KE2) // NS)
    def _(j):
        base = (sid + j * NS) * KE2
        d_src = pltpu.async_copy(srcs_hbm.at[pl.ds(cid * E + base, KE2)],
                                 srcb, isem)
        d_dst = pltpu.async_copy(dst_hbm.at[pl.ds(base, KE2)], dstb, isem)
        d_ee = pltpu.async_copy(ee_hbm.at[pl.ds(base * HEADS, KE2 * HEADS)],
                                eeb, isem)
        d_src.wait()
        d_dst.wait()
        d_ee.wait()
        d_rows = pltpu.async_copy(hh2_hbm.at[srcb], rows, gsem)
        d_iv = pltpu.async_copy(invd_hbm.at[dstb], ivb, vsem)
        d_iv.wait()
        for i in range(KE2 // LANES):
            r16 = i * LANES + io
            for hd in range(HEADS):
                hv = jnp.full((LANES,), hd, i32)
                ee = plsc.load_gather(eeb, [r16 * HEADS + hv])
                iv = plsc.load_gather(ivb, [r16, hv])
                plsc.store_scatter(wb, [r16 * HEADS + hv], ee * iv)
        d_rows.wait()

        @pl.loop(0, KE2)
        def _(r):
            rv = jnp.zeros((LANES,), i32) + r * HEADS
            w32 = []
            for hd in range(HEADS):
                hv = jnp.full((LANES,), hd, i32)
                wv = plsc.load_gather(wb, [rv + hv])
                w32.append(plsc.pack(wv, wv,
                                     format=plsc.PackFormat.INTERLEAVED))
            for g in range(4):
                acc = jnp.zeros((2 * LANES,), bf16)
                for hd in range(HEADS):
                    off = hd * 64 + g * LANES
                    ri = rows[r, off // 128, pl.ds(off % 128, LANES)]
                    acc = acc + w32[hd] * plsc.bitcast(ri, bf16)
                ev, od = plsc.unpack(acc, format=plsc.PackFormat.INTERLEAVED)
                outst[r, pl.ds(g * 32, LANES)] = ev
                outst[r, pl.ds(g * 32 + LANES, LANES)] = od

        pltpu.sync_copy(outst, gacc_sp.at[dstb], add=True)

    plsc.subcore_barrier()
    pltpu.sync_copy(gacc_sp.at[pl.ds(sid * NPS, NPS)],
                    out_hbm.at[cid, pl.ds(sid * NPS, NPS)])


def _sc_gat2(hh2b, src2, dst, ee4, invd128):
    return pl.kernel(
        _gat2_body,
        out_type=jax.ShapeDtypeStruct((NC, NP, 128), f32),
        mesh=_MESH,
        scratch_types=[
            pltpu.VMEM((KE2,), i32),
            pltpu.VMEM((KE2,), i32),
            pltpu.VMEM((KE2 * HEADS,), f32),
            pltpu.VMEM((KE2 * HEADS,), f32),
            pltpu.VMEM((KE2, 128), f32),
            pltpu.VMEM((KE2, 2, 128), i32),
            pltpu.VMEM((KE2, 128), f32),
            pltpu.VMEM((32, 128), f32),
            pltpu.VMEM_SHARED((NP, 128), f32),
            pltpu.SemaphoreType.DMA,
            pltpu.SemaphoreType.DMA,
            pltpu.SemaphoreType.DMA,
        ],
        compiler_params=_SC_PARAMS,
    )(hh2b, src2, dst, ee4, invd128)


def _mm_body(x_ref, w_ref, o_ref):
    o_ref[...] = jnp.dot(x_ref[...], w_ref[...],
                         preferred_element_type=f32)


def _tc_matmul(x, w):
    m, k = x.shape
    _, n = w.shape
    return pl.pallas_call(
        _mm_body,
        grid=(m // BN,),
        in_specs=[pl.BlockSpec((BN, k), lambda i: (i, 0)),
                  pl.BlockSpec((k, n), lambda i: (0, 0))],
        out_specs=pl.BlockSpec((BN, n), lambda i: (i, 0)),
        out_shape=jax.ShapeDtypeStruct((m, n), f32),
    )(x, w)


def _prescale_body(hx_ref, h0_ref, h1_ref, hp_ref, dinv_ref, degn1_ref):
    degn = h0_ref[...] + h1_ref[...]
    deg = degn + 1.0
    dinv = lax.rsqrt(deg)
    hp_ref[...] = hx_ref[...] * dinv
    dinv_ref[...] = dinv
    degn1_ref[...] = jnp.maximum(degn, 1.0)


def _tc_prescale(hx, h0, h1):
    return pl.pallas_call(
        _prescale_body,
        grid=(N // BN,),
        in_specs=[pl.BlockSpec((BN, HID), lambda i: (i, 0)),
                  pl.BlockSpec((BN, 1), lambda i: (i, 0)),
                  pl.BlockSpec((BN, 1), lambda i: (i, 0))],
        out_specs=[pl.BlockSpec((BN, HID), lambda i: (i, 0)),
                   pl.BlockSpec((BN, 1), lambda i: (i, 0)),
                   pl.BlockSpec((BN, 1), lambda i: (i, 0))],
        out_shape=[jax.ShapeDtypeStruct((N, HID), f32),
                   jax.ShapeDtypeStruct((N, 1), f32),
                   jax.ShapeDtypeStruct((N, 1), f32)],
    )(hx, h0, h1)


def _gcn_gat_body(agg_ref, hp_ref, dinv_ref, bg_ref, wgat_ref, amat_ref,
                  hh_ref, asad_ref, macc_ref):
    i = pl.program_id(0)
    h = jnp.maximum(dinv_ref[...] * (agg_ref[...] + hp_ref[...])
                    + bg_ref[...], 0.0)
    hh = jnp.dot(h, wgat_ref[...], preferred_element_type=f32)
    hh_ref[...] = hh
    asad = jnp.dot(hh, amat_ref[...], preferred_element_type=f32)
    asad_ref[...] = jnp.concatenate(
        [asad, jnp.zeros((asad.shape[0], 120), f32)], axis=1)
    mx = jnp.max(asad, axis=0, keepdims=True)
    mx8 = jnp.broadcast_to(mx, (8, 8))
    prev = jnp.where(i == 0, jnp.full((8, 8), -jnp.inf, f32), macc_ref[...])
    macc_ref[...] = jnp.maximum(prev, mx8)


def _tc_gcn_gat(aggcat, hp, dinv, b_gcn, W_gat, Amat):
    return pl.pallas_call(
        _gcn_gat_body,
        grid=(N // BN,),
        in_specs=[pl.BlockSpec((BN, HID), lambda i: (i, 0)),
                  pl.BlockSpec((BN, HID), lambda i: (i, 0)),
                  pl.BlockSpec((BN, 1), lambda i: (i, 0)),
                  pl.BlockSpec((1, HID), lambda i: (0, 0)),
                  pl.BlockSpec((HID, HEADS * HID), lambda i: (0, 0)),
                  pl.BlockSpec((HEADS * HID, 8), lambda i: (0, 0))],
        out_specs=[pl.BlockSpec((BN, HEADS * HID), lambda i: (i, 0)),
                   pl.BlockSpec((BN, 128), lambda i: (i, 0)),
                   pl.BlockSpec((8, 8), lambda i: (0, 0))],
        out_shape=[jax.ShapeDtypeStruct((N, HEADS * HID), f32),
                   jax.ShapeDtypeStruct((N, 128), f32),
                   jax.ShapeDtypeStruct((8, 8), f32)],
    )(aggcat, hp, dinv, b_gcn, W_gat, Amat)


def _denom_body(den_ref, asad_ref, m4_ref, hh_ref, invd_ref, gself_ref):
    den = den_ref[0, :, :HEADS] + den_ref[1, :, :HEADS]
    es = asad_ref[:, :HEADS] + asad_ref[:, HEADS:]
    lr = jnp.maximum(es, 0.2 * es)
    se = jnp.exp(lr - m4_ref[...])
    dtot = den + se
    invd = 1.0 / (4.0 * dtot)
    invd_ref[...] = jnp.concatenate(
        [invd, jnp.zeros((invd.shape[0], 124), f32)], axis=1)
    acc = jnp.zeros_like(gself_ref)
    for hd in range(HEADS):
        coef = se[:, hd:hd + 1] * invd[:, hd:hd + 1]
        acc = acc + coef * hh_ref[:, hd * HID:(hd + 1) * HID]
    gself_ref[...] = acc


def _tc_denom(denp, asad, M4r, hh):
    return pl.pallas_call(
        _denom_body,
        grid=(N // BN,),
        in_specs=[pl.BlockSpec((NC, BN, 128), lambda i: (0, i, 0)),
                  pl.BlockSpec((BN, 8), lambda i: (i, 0)),
                  pl.BlockSpec((1, HEADS), lambda i: (0, 0)),
                  pl.BlockSpec((BN, HEADS * HID), lambda i: (i, 0))],
        out_specs=[pl.BlockSpec((BN, 128), lambda i: (i, 0)),
                   pl.BlockSpec((BN, HID), lambda i: (i, 0))],
        out_shape=[jax.ShapeDtypeStruct((N, 128), f32),
                   jax.ShapeDtypeStruct((N, HID), f32)],
    )(denp, asad, M4r, hh)


def _gat_fin_body(graw_ref, gself_ref, bgat_ref, g_ref):
    g_ref[...] = jnp.maximum(graw_ref[...] + gself_ref[...] + bgat_ref[...],
                             0.0)


def _tc_gat_fin(grawcat, gself, b_gat):
    return pl.pallas_call(
        _gat_fin_body,
        grid=(N // BN,),
        in_specs=[pl.BlockSpec((BN, HID), lambda i: (i, 0)),
                  pl.BlockSpec((BN, HID), lambda i: (i, 0)),
                  pl.BlockSpec((1, HID), lambda i: (0, 0))],
        out_specs=pl.BlockSpec((BN, HID), lambda i: (i, 0)),
        out_shape=jax.ShapeDtypeStruct((N, HID), f32),
    )(grawcat, gself, b_gat)


def _head_body(nsum_ref, degn1_ref, g_ref, wl_ref, wr_ref, bs_ref,
               wc1_ref, bc1_ref, wc2_ref, bc2_ref, o_ref):
    neigh = nsum_ref[...] / degn1_ref[...]
    s = (jnp.dot(neigh, wl_ref[...], preferred_element_type=f32)
         + jnp.dot(g_ref[...], wr_ref[...], preferred_element_type=f32)
         + bs_ref[...])
    c = jnp.maximum(jnp.dot(s, wc1_ref[...], preferred_element_type=f32)
                    + bc1_ref[...], 0.0)
    logits = jnp.dot(c, wc2_ref[...], preferred_element_type=f32) + bc2_ref[...]
    o_ref[...] = jax.nn.sigmoid(logits)


def _tc_head(nsumcat, degn1, g, W_sage_l, W_sage_r, bs, W_c1, bc1, W_c2p, bc2):
    return pl.pallas_call(
        _head_body,
        grid=(N // BN,),
        in_specs=[pl.BlockSpec((BN, HID), lambda i: (i, 0)),
                  pl.BlockSpec((BN, 1), lambda i: (i, 0)),
                  pl.BlockSpec((BN, HID), lambda i: (i, 0)),
                  pl.BlockSpec((HID, HID), lambda i: (0, 0)),
                  pl.BlockSpec((HID, HID), lambda i: (0, 0)),
                  pl.BlockSpec((1, HID), lambda i: (0, 0)),
                  pl.BlockSpec((HID, HID // 2), lambda i: (0, 0)),
                  pl.BlockSpec((1, HID // 2), lambda i: (0, 0)),
                  pl.BlockSpec((HID // 2, 8), lambda i: (0, 0)),
                  pl.BlockSpec((1, 8), lambda i: (0, 0))],
        out_specs=pl.BlockSpec((BN, 8), lambda i: (i, 0)),
        out_shape=jax.ShapeDtypeStruct((N, 8), f32),
    )(nsumcat, degn1, g, W_sage_l, W_sage_r, bs, W_c1, bc1, W_c2p, bc2)


def kernel(x, edge_index, W_gcn, b_gcn, W_gat, att_src, att_dst, b_gat,
           W_sage_l, W_sage_r, b_sage, W_c1, b_c1, W_c2, b_c2):
    src = edge_index[0]
    dst = edge_index[1]
    src2 = jnp.concatenate([src, src + N])

    hist = _sc_hist(dst)
    hx = _tc_matmul(x, W_gcn)
    h0 = hist[0, :N, :1]
    h1 = hist[1, :N, :1]
    hp, dinv, degn1 = _tc_prescale(hx, h0, h1)
    hp2 = jnp.concatenate([hp[:, :128], hp[:, 128:]], axis=0)
    agg = _sc_agg(hp2, src2, dst)
    aggcat = jnp.concatenate([agg[0, :N], agg[1, :N]], axis=1)

    eye = jnp.eye(HEADS, dtype=f32)
    As = (att_src[:, :, None] * eye[:, None, :]).reshape(HEADS * HID, HEADS)
    Ad = (att_dst[:, :, None] * eye[:, None, :]).reshape(HEADS * HID, HEADS)
    Amat = jnp.concatenate([As, Ad], axis=1)

    hh, asad, macc = _tc_gcn_gat(aggcat, hp, dinv, b_gcn.reshape(1, HID),
                                 W_gat, Amat)
    ms = macc[0, :HEADS]
    md = macc[0, HEADS:]
    msum = ms + md
    M4 = jnp.maximum(msum, 0.2 * msum)
    mrep = jnp.broadcast_to(M4[:, None], (HEADS, 16))

    ee4, denp = _sc_gat1(src, dst, asad, mrep)
    invd128, gself = _tc_denom(denp, asad[:, :8], M4.reshape(1, HEADS), hh)

    hh4 = hh.reshape(N, HEADS, HID)
    hh2b = jnp.concatenate([hh4[:, :, :128], hh4[:, :, 128:]],
                           axis=0).astype(jnp.bfloat16)
    hh2i = jax.lax.bitcast_convert_type(
        hh2b.reshape(2 * N, 256, 2), i32).reshape(2 * N, 2, 128)
    graw = _sc_gat2(hh2i, src2, dst, ee4, invd128)
    grawcat = jnp.concatenate([graw[0, :N], graw[1, :N]], axis=1)
    perm = [(c // 32) * 32 + ((c % 32) // 2 if c % 2 == 0
                              else 16 + (c % 32) // 2) for c in range(HID)]
    grawcat = grawcat[:, jnp.array(perm, dtype=i32)]
    g = _tc_gat_fin(grawcat, gself, b_gat.reshape(1, HID))

    g2 = jnp.concatenate([g[:, :128], g[:, 128:]], axis=0)
    nsum = _sc_agg(g2, src2, dst)
    nsumcat = jnp.concatenate([nsum[0, :N], nsum[1, :N]], axis=1)

    W_c2p = jnp.concatenate([W_c2, jnp.zeros((HID // 2, 7), f32)], axis=1)
    bc2p = jnp.concatenate([b_c2, jnp.zeros((7,), f32)]).reshape(1, 8)
    out8 = _tc_head(nsumcat, degn1, g, W_sage_l, W_sage_r,
                    b_sage.reshape(1, HID), W_c1, b_c1.reshape(1, HID // 2),
                    W_c2p, bc2p)
    return out8[:, :1]


# speedup vs baseline: 1.7525x; 1.0175x over previous
import dataclasses

import jax
import jax.numpy as jnp
from jax import lax
from jax.experimental import pallas as pl
from jax.experimental.pallas import tpu as pltpu
from jax.experimental.pallas import tpu_sc as plsc

N = 10000
E = 160000
D_IN = 256
HID = 256
HEADS = 4

NC, NS, LANES = 2, 16, 16
NW = NC * NS
NP = 10240
NPS = NP // NS
BN = 1000

KA = 40
KC = 40
KE1 = 80
KE2 = 80

_MESH = plsc.VectorSubcoreMesh(
    core_axis_name="c", subcore_axis_name="s", num_cores=NC, num_subcores=NS)

f32 = jnp.float32
i32 = jnp.int32

_SC_PARAMS = pltpu.CompilerParams()
if "needs_layout_passes" in pltpu.CompilerParams.__dataclass_fields__:
    _SC_PARAMS = dataclasses.replace(_SC_PARAMS, needs_layout_passes=False)


def _fill_zero(buf):
    rows, cols = buf.shape
    z = jnp.zeros((LANES,), f32)
    for r in range(rows):
        for g in range(cols // LANES):
            buf.at[r][pl.ds(g * LANES, LANES)] = z


def _zero_spmem(sp, zbuf, sid):
    zr = zbuf.shape[0]
    reps = NPS // zr

    @pl.loop(0, reps)
    def _(i):
        pltpu.sync_copy(zbuf, sp.at[pl.ds(sid * NPS + i * zr, zr)])


def _hist_body(dst_hbm, out_hbm, idxv, onesv, zbuf, hist_sp):
    cid = lax.axis_index("c")
    sid = lax.axis_index("s")
    w = sid * NC + cid
    ii = lax.iota(i32, LANES)
    row1 = jnp.where(ii == 0, 1.0, 0.0).astype(f32)
    _fill_zero(onesv)
    for r in range(KA):
        onesv.at[r][pl.ds(0, LANES)] = row1
    _fill_zero(zbuf)
    _zero_spmem(hist_sp, zbuf, sid)
    plsc.subcore_barrier()

    @pl.loop(0, (E // KA) // NW)
    def _(j):
        base = (w + j * NW) * KA
        pltpu.sync_copy(dst_hbm.at[pl.ds(base, KA)], idxv)
        pltpu.sync_copy(onesv, hist_sp.at[idxv], add=True)

    plsc.subcore_barrier()
    pltpu.sync_copy(hist_sp.at[pl.ds(sid * NPS, NPS)],
                    out_hbm.at[cid, pl.ds(sid * NPS, NPS)])


def _sc_hist(dst):
    return pl.kernel(
        _hist_body,
        out_type=jax.ShapeDtypeStruct((NC, NP, 128), f32),
        mesh=_MESH,
        scratch_types=[
            pltpu.VMEM((KA,), i32),
            pltpu.VMEM((KA, 128), f32),
            pltpu.VMEM((32, 128), f32),
            pltpu.VMEM_SHARED((NP, 128), f32),
        ],
    )(dst)


def _agg_body(tab_hbm, srcs_hbm, dst_hbm, out_hbm, gidx, sidx, rows, zbuf,
              acc_sp, gsem, ssem):
    cid = lax.axis_index("c")
    sid = lax.axis_index("s")
    _fill_zero(zbuf)
    _zero_spmem(acc_sp, zbuf, sid)
    plsc.subcore_barrier()

    P = 5

    @pl.loop(0, (E // KC) // NS // P)
    def _(jj):
        descs = []
        for p in range(P):
            base = (sid + (jj * P + p) * NS) * KC
            pltpu.sync_copy(srcs_hbm.at[pl.ds(cid * E + base, KC)],
                            gidx.at[p])
            pltpu.sync_copy(dst_hbm.at[pl.ds(base, KC)], sidx.at[p])
            descs.append(pltpu.async_copy(tab_hbm.at[gidx.at[p]],
                                          rows.at[p], gsem.at[p]))
        sdescs = []
        for p in range(P):
            descs[p].wait()
            sdescs.append(pltpu.async_copy(rows.at[p],
                                           acc_sp.at[sidx.at[p]], ssem.at[p],
                                           add=True))
        for p in range(P):
            sdescs[p].wait()

    plsc.subcore_barrier()
    pltpu.sync_copy(acc_sp.at[pl.ds(sid * NPS, NPS)],
                    out_hbm.at[cid, pl.ds(sid * NPS, NPS)])


def _sc_agg(tab2, src2, dst):
    return pl.kernel(
        _agg_body,
        out_type=jax.ShapeDtypeStruct((NC, NP, 128), f32),
        mesh=_MESH,
        scratch_types=[
            pltpu.VMEM((5, KC), i32),
            pltpu.VMEM((5, KC), i32),
            pltpu.VMEM((5, KC, 128), f32),
            pltpu.VMEM((32, 128), f32),
            pltpu.VMEM_SHARED((NP, 128), f32),
            pltpu.SemaphoreType.DMA((5,)),
            pltpu.SemaphoreType.DMA((5,)),
        ],
    )(tab2, src2, dst)


def _gat1_body(src_hbm, dst_hbm, asad_hbm, mrep_hbm, ee_hbm, den_hbm,
               mrep_v, srcb, dstb, asb, adb, st80, eeb, zbuf, den_sp,
               asem, bsem):
    cid = lax.axis_index("c")
    sid = lax.axis_index("s")
    pltpu.sync_copy(mrep_hbm, mrep_v)
    _fill_zero(st80)
    _fill_zero(zbuf)
    _zero_spmem(den_sp, zbuf, sid)
    plsc.subcore_barrier()

    io = lax.iota(i32, LANES)
    w = sid * NC + cid
    nch = E // KE1

    @pl.loop(0, (nch + NW - 1) // NW)
    def _(j):
        c = w + j * NW

        @pl.when(c < nch)
        def _():
            base = c * KE1
            pltpu.sync_copy(src_hbm.at[pl.ds(base, KE1)], srcb)
            pltpu.sync_copy(dst_hbm.at[pl.ds(base, KE1)], dstb)
            da = pltpu.async_copy(asad_hbm.at[srcb], asb, asem)
            db = pltpu.async_copy(asad_hbm.at[dstb], adb, bsem)
            da.wait()
            db.wait()
            for i in range(KE1 // LANES):
                r16 = i * LANES + io
                for hd in range(HEADS):
                    hv = jnp.full((LANES,), hd, i32)
                    av = plsc.load_gather(asb, [r16, hv])
                    bv = plsc.load_gather(adb, [r16, hv + HEADS])
                    ev = av + bv
                    lr = jnp.maximum(ev, 0.2 * ev)
                    fv = jnp.exp(lr - mrep_v.at[hd][pl.ds(0, LANES)])
                    plsc.store_scatter(st80, [r16, hv], fv)
                    plsc.store_scatter(eeb, [r16 * HEADS + hv], fv)
            pltpu.sync_copy(st80, den_sp.at[dstb], add=True)
            pltpu.sync_copy(eeb, ee_hbm.at[pl.ds(base * HEADS, KE1 * HEADS)])

    plsc.subcore_barrier()
    pltpu.sync_copy(den_sp.at[pl.ds(sid * NPS, NPS)],
                    den_hbm.at[cid, pl.ds(sid * NPS, NPS)])


def _sc_gat1(src, dst, asad128, mrep):
    return pl.kernel(
        _gat1_body,
        out_type=[jax.ShapeDtypeStruct((E * HEADS,), f32),
                  jax.ShapeDtypeStruct((NC, NP, 128), f32)],
        mesh=_MESH,
        scratch_types=[
            pltpu.VMEM((HEADS, 16), f32),
            pltpu.VMEM((KE1,), i32),
            pltpu.VMEM((KE1,), i32),
            pltpu.VMEM((KE1, 128), f32),
            pltpu.VMEM((KE1, 128), f32),
            pltpu.VMEM((KE1, 128), f32),
            pltpu.VMEM((KE1 * HEADS,), f32),
            pltpu.VMEM((32, 128), f32),
            pltpu.VMEM_SHARED((NP, 128), f32),
            pltpu.SemaphoreType.DMA,
            pltpu.SemaphoreType.DMA,
        ],
        compiler_params=_SC_PARAMS,
    )(src, dst, asad128, mrep)


def _gat2_body(hh2_hbm, srcs_hbm, dst_hbm, ee_hbm, invd_hbm, out_hbm,
               srcb2, dstb, eeb, wb, ivb, rowsA, rowsB, outst, zbuf, gacc_sp,
               isem, gsemA, gsemB, vsem):
    cid = lax.axis_index("c")
    sid = lax.axis_index("s")
    _fill_zero(zbuf)
    _zero_spmem(gacc_sp, zbuf, sid)
    plsc.subcore_barrier()

    io = lax.iota(i32, LANES)
    bf16 = jnp.bfloat16
    H2 = KE2 // 2

    @pl.loop(0, (E // KE2) // NS)
    def _(j):
        base = (sid + j * NS) * KE2
        d_s0 = pltpu.async_copy(srcs_hbm.at[pl.ds(cid * E + base, H2)],
                                srcb2.at[0], isem.at[0])
        d_s1 = pltpu.async_copy(srcs_hbm.at[pl.ds(cid * E + base + H2, H2)],
                                srcb2.at[1], isem.at[1])
        d_dst = pltpu.async_copy(dst_hbm.at[pl.ds(base, KE2)], dstb,
                                 isem.at[2])
        d_ee = pltpu.async_copy(ee_hbm.at[pl.ds(base * HEADS, KE2 * HEADS)],
                                eeb, isem.at[3])
        d_s0.wait()
        dA = pltpu.async_copy(hh2_hbm.at[srcb2.at[0]], rowsA, gsemA)
        d_s1.wait()
        dB = pltpu.async_copy(hh2_hbm.at[srcb2.at[1]], rowsB, gsemB)
        d_dst.wait()
        d_iv = pltpu.async_copy(invd_hbm.at[dstb], ivb, vsem)
        d_ee.wait()
        d_iv.wait()
        for i in range(KE2 // LANES):
            r16 = i * LANES + io
            for hd in range(HEADS):
                hv = jnp.full((LANES,), hd, i32)
                ee = plsc.load_gather(eeb, [r16 * HEADS + hv])
                iv = plsc.load_gather(ivb, [r16, hv])
                plsc.store_scatter(wb, [r16 * HEADS + hv], ee * iv)

        def _edge_block(rows, roff):
            @pl.loop(0, H2)
            def _(r):
                rv = jnp.zeros((LANES,), i32) + (r + roff) * HEADS
                w32 = []
                for hd in range(HEADS):
                    hv = jnp.full((LANES,), hd, i32)
                    wv = plsc.load_gather(wb, [rv + hv])
                    w32.append(plsc.pack(wv, wv,
                                         format=plsc.PackFormat.INTERLEAVED))
                for g in range(4):
                    acc = jnp.zeros((2 * LANES,), bf16)
                    for hd in range(HEADS):
                        off = hd * 64 + g * LANES
                        ri = rows[r, off // 128, pl.ds(off % 128, LANES)]
                        acc = acc + w32[hd] * plsc.bitcast(ri, bf16)
                    ev, od = plsc.unpack(acc,
                                         format=plsc.PackFormat.INTERLEAVED)
                    outst[r + roff, pl.ds(g * 32, LANES)] = ev
                    outst[r + roff, pl.ds(g * 32 + LANES, LANES)] = od

        dA.wait()
        _edge_block(rowsA, 0)
        dB.wait()
        _edge_block(rowsB, H2)

        pltpu.sync_copy(outst, gacc_sp.at[dstb], add=True)

    plsc.subcore_barrier()
    pltpu.sync_copy(gacc_sp.at[pl.ds(sid * NPS, NPS)],
                    out_hbm.at[cid, pl.ds(sid * NPS, NPS)])


def _sc_gat2(hh2b, src2, dst, ee4, invd128):
    return pl.kernel(
        _gat2_body,
        out_type=jax.ShapeDtypeStruct((NC, NP, 128), f32),
        mesh=_MESH,
        scratch_types=[
            pltpu.VMEM((2, KE2 // 2), i32),
            pltpu.VMEM((KE2,), i32),
            pltpu.VMEM((KE2 * HEADS,), f32),
            pltpu.VMEM((KE2 * HEADS,), f32),
            pltpu.VMEM((KE2, 128), f32),
            pltpu.VMEM((KE2 // 2, 2, 128), i32),
            pltpu.VMEM((KE2 // 2, 2, 128), i32),
            pltpu.VMEM((KE2, 128), f32),
            pltpu.VMEM((32, 128), f32),
            pltpu.VMEM_SHARED((NP, 128), f32),
            pltpu.SemaphoreType.DMA((4,)),
            pltpu.SemaphoreType.DMA,
            pltpu.SemaphoreType.DMA,
            pltpu.SemaphoreType.DMA,
        ],
        compiler_params=_SC_PARAMS,
    )(hh2b, src2, dst, ee4, invd128)


def _mm_body(x_ref, w_ref, o_ref):
    o_ref[...] = jnp.dot(x_ref[...], w_ref[...],
                         preferred_element_type=f32)


def _tc_matmul(x, w):
    m, k = x.shape
    _, n = w.shape
    return pl.pallas_call(
        _mm_body,
        grid=(m // BN,),
        in_specs=[pl.BlockSpec((BN, k), lambda i: (i, 0)),
                  pl.BlockSpec((k, n), lambda i: (0, 0))],
        out_specs=pl.BlockSpec((BN, n), lambda i: (i, 0)),
        out_shape=jax.ShapeDtypeStruct((m, n), f32),
    )(x, w)


def _prescale_body(hx_ref, h0_ref, h1_ref, hp_ref, dinv_ref, degn1_ref):
    degn = h0_ref[...] + h1_ref[...]
    deg = degn + 1.0
    dinv = lax.rsqrt(deg)
    hp_ref[...] = hx_ref[...] * dinv
    dinv_ref[...] = dinv
    degn1_ref[...] = jnp.maximum(degn, 1.0)


def _tc_prescale(hx, h0, h1):
    return pl.pallas_call(
        _prescale_body,
        grid=(N // BN,),
        in_specs=[pl.BlockSpec((BN, HID), lambda i: (i, 0)),
                  pl.BlockSpec((BN, 1), lambda i: (i, 0)),
                  pl.BlockSpec((BN, 1), lambda i: (i, 0))],
        out_specs=[pl.BlockSpec((BN, HID), lambda i: (i, 0)),
                   pl.BlockSpec((BN, 1), lambda i: (i, 0)),
                   pl.BlockSpec((BN, 1), lambda i: (i, 0))],
        out_shape=[jax.ShapeDtypeStruct((N, HID), f32),
                   jax.ShapeDtypeStruct((N, 1), f32),
                   jax.ShapeDtypeStruct((N, 1), f32)],
    )(hx, h0, h1)


def _gcn_gat_body(agg_ref, hp_ref, dinv_ref, bg_ref, wgat_ref, amat_ref,
                  hh_ref, asad_ref, macc_ref):
    i = pl.program_id(0)
    h = jnp.maximum(dinv_ref[...] * (agg_ref[...] + hp_ref[...])
                    + bg_ref[...], 0.0)
    hh = jnp.dot(h, wgat_ref[...], preferred_element_type=f32)
    hh_ref[...] = hh
    asad = jnp.dot(hh, amat_ref[...], preferred_element_type=f32)
    asad_ref[...] = jnp.concatenate(
        [asad, jnp.zeros((asad.shape[0], 120), f32)], axis=1)
    mx = jnp.max(asad, axis=0, keepdims=True)
    mx8 = jnp.broadcast_to(mx, (8, 8))
    prev = jnp.where(i == 0, jnp.full((8, 8), -jnp.inf, f32), macc_ref[...])
    macc_ref[...] = jnp.maximum(prev, mx8)


def _tc_gcn_gat(aggcat, hp, dinv, b_gcn, W_gat, Amat):
    return pl.pallas_call(
        _gcn_gat_body,
        grid=(N // BN,),
        in_specs=[pl.BlockSpec((BN, HID), lambda i: (i, 0)),
                  pl.BlockSpec((BN, HID), lambda i: (i, 0)),
                  pl.BlockSpec((BN, 1), lambda i: (i, 0)),
                  pl.BlockSpec((1, HID), lambda i: (0, 0)),
                  pl.BlockSpec((HID, HEADS * HID), lambda i: (0, 0)),
                  pl.BlockSpec((HEADS * HID, 8), lambda i: (0, 0))],
        out_specs=[pl.BlockSpec((BN, HEADS * HID), lambda i: (i, 0)),
                   pl.BlockSpec((BN, 128), lambda i: (i, 0)),
                   pl.BlockSpec((8, 8), lambda i: (0, 0))],
        out_shape=[jax.ShapeDtypeStruct((N, HEADS * HID), f32),
                   jax.ShapeDtypeStruct((N, 128), f32),
                   jax.ShapeDtypeStruct((8, 8), f32)],
    )(aggcat, hp, dinv, b_gcn, W_gat, Amat)


def _denom_body(den_ref, asad_ref, m4_ref, hh_ref, invd_ref, gself_ref):
    den = den_ref[0, :, :HEADS] + den_ref[1, :, :HEADS]
    es = asad_ref[:, :HEADS] + asad_ref[:, HEADS:]
    lr = jnp.maximum(es, 0.2 * es)
    se = jnp.exp(lr - m4_ref[...])
    dtot = den + se
    invd = 1.0 / (4.0 * dtot)
    invd_ref[...] = jnp.concatenate(
        [invd, jnp.zeros((invd.shape[0], 124), f32)], axis=1)
    acc = jnp.zeros_like(gself_ref)
    for hd in range(HEADS):
        coef = se[:, hd:hd + 1] * invd[:, hd:hd + 1]
        acc = acc + coef * hh_ref[:, hd * HID:(hd + 1) * HID]
    gself_ref[...] = acc


def _tc_denom(denp, asad, M4r, hh):
    return pl.pallas_call(
        _denom_body,
        grid=(N // BN,),
        in_specs=[pl.BlockSpec((NC, BN, 128), lambda i: (0, i, 0)),
                  pl.BlockSpec((BN, 8), lambda i: (i, 0)),
                  pl.BlockSpec((1, HEADS), lambda i: (0, 0)),
                  pl.BlockSpec((BN, HEADS * HID), lambda i: (i, 0))],
        out_specs=[pl.BlockSpec((BN, 128), lambda i: (i, 0)),
                   pl.BlockSpec((BN, HID), lambda i: (i, 0))],
        out_shape=[jax.ShapeDtypeStruct((N, 128), f32),
                   jax.ShapeDtypeStruct((N, HID), f32)],
    )(denp, asad, M4r, hh)


def _gat_fin_body(graw_ref, gself_ref, bgat_ref, g_ref):
    g_ref[...] = jnp.maximum(graw_ref[...] + gself_ref[...] + bgat_ref[...],
                             0.0)


def _tc_gat_fin(grawcat, gself, b_gat):
    return pl.pallas_call(
        _gat_fin_body,
        grid=(N // BN,),
        in_specs=[pl.BlockSpec((BN, HID), lambda i: (i, 0)),
                  pl.BlockSpec((BN, HID), lambda i: (i, 0)),
                  pl.BlockSpec((1, HID), lambda i: (0, 0))],
        out_specs=pl.BlockSpec((BN, HID), lambda i: (i, 0)),
        out_shape=jax.ShapeDtypeStruct((N, HID), f32),
    )(grawcat, gself, b_gat)


def _head_body(nsum_ref, degn1_ref, g_ref, wl_ref, wr_ref, bs_ref,
               wc1_ref, bc1_ref, wc2_ref, bc2_ref, o_ref):
    neigh = nsum_ref[...] / degn1_ref[...]
    s = (jnp.dot(neigh, wl_ref[...], preferred_element_type=f32)
         + jnp.dot(g_ref[...], wr_ref[...], preferred_element_type=f32)
         + bs_ref[...])
    c = jnp.maximum(jnp.dot(s, wc1_ref[...], preferred_element_type=f32)
                    + bc1_ref[...], 0.0)
    logits = jnp.dot(c, wc2_ref[...], preferred_element_type=f32) + bc2_ref[...]
    o_ref[...] = jax.nn.sigmoid(logits)


def _tc_head(nsumcat, degn1, g, W_sage_l, W_sage_r, bs, W_c1, bc1, W_c2p, bc2):
    return pl.pallas_call(
        _head_body,
        grid=(N // BN,),
        in_specs=[pl.BlockSpec((BN, HID), lambda i: (i, 0)),
                  pl.BlockSpec((BN, 1), lambda i: (i, 0)),
                  pl.BlockSpec((BN, HID), lambda i: (i, 0)),
                  pl.BlockSpec((HID, HID), lambda i: (0, 0)),
                  pl.BlockSpec((HID, HID), lambda i: (0, 0)),
                  pl.BlockSpec((1, HID), lambda i: (0, 0)),
                  pl.BlockSpec((HID, HID // 2), lambda i: (0, 0)),
                  pl.BlockSpec((1, HID // 2), lambda i: (0, 0)),
                  pl.BlockSpec((HID // 2, 8), lambda i: (0, 0)),
                  pl.BlockSpec((1, 8), lambda i: (0, 0))],
        out_specs=pl.BlockSpec((BN, 8), lambda i: (i, 0)),
        out_shape=jax.ShapeDtypeStruct((N, 8), f32),
    )(nsumcat, degn1, g, W_sage_l, W_sage_r, bs, W_c1, bc1, W_c2p, bc2)


def kernel(x, edge_index, W_gcn, b_gcn, W_gat, att_src, att_dst, b_gat,
           W_sage_l, W_sage_r, b_sage, W_c1, b_c1, W_c2, b_c2):
    src = edge_index[0]
    dst = edge_index[1]
    src2 = jnp.concatenate([src, src + N])

    hist = _sc_hist(dst)
    hx = _tc_matmul(x, W_gcn)
    h0 = hist[0, :N, :1]
    h1 = hist[1, :N, :1]
    hp, dinv, degn1 = _tc_prescale(hx, h0, h1)
    hp2 = jnp.concatenate([hp[:, :128], hp[:, 128:]], axis=0)
    agg = _sc_agg(hp2, src2, dst)
    aggcat = jnp.concatenate([agg[0, :N], agg[1, :N]], axis=1)

    eye = jnp.eye(HEADS, dtype=f32)
    As = (att_src[:, :, None] * eye[:, None, :]).reshape(HEADS * HID, HEADS)
    Ad = (att_dst[:, :, None] * eye[:, None, :]).reshape(HEADS * HID, HEADS)
    Amat = jnp.concatenate([As, Ad], axis=1)

    hh, asad, macc = _tc_gcn_gat(aggcat, hp, dinv, b_gcn.reshape(1, HID),
                                 W_gat, Amat)
    ms = macc[0, :HEADS]
    md = macc[0, HEADS:]
    msum = ms + md
    M4 = jnp.maximum(msum, 0.2 * msum)
    mrep = jnp.broadcast_to(M4[:, None], (HEADS, 16))

    ee4, denp = _sc_gat1(src, dst, asad, mrep)
    invd128, gself = _tc_denom(denp, asad[:, :8], M4.reshape(1, HEADS), hh)

    hh4 = hh.reshape(N, HEADS, HID)
    hh2b = jnp.concatenate([hh4[:, :, :128], hh4[:, :, 128:]],
                           axis=0).astype(jnp.bfloat16)
    hh2i = jax.lax.bitcast_convert_type(
        hh2b.reshape(2 * N, 256, 2), i32).reshape(2 * N, 2, 128)
    graw = _sc_gat2(hh2i, src2, dst, ee4, invd128)
    grawcat = jnp.concatenate([graw[0, :N], graw[1, :N]], axis=1)
    perm = [(c // 32) * 32 + ((c % 32) // 2 if c % 2 == 0
                              else 16 + (c % 32) // 2) for c in range(HID)]
    grawcat = grawcat[:, jnp.array(perm, dtype=i32)]
    g = _tc_gat_fin(grawcat, gself, b_gat.reshape(1, HID))

    g2 = jnp.concatenate([g[:, :128], g[:, 128:]], axis=0)
    nsum = _sc_agg(g2, src2, dst)
    nsumcat = jnp.concatenate([nsum[0, :N], nsum[1, :N]], axis=1)

    W_c2p = jnp.concatenate([W_c2, jnp.zeros((HID // 2, 7), f32)], axis=1)
    bc2p = jnp.concatenate([b_c2, jnp.zeros((7,), f32)]).reshape(1, 8)
    out8 = _tc_head(nsumcat, degn1, g, W_sage_l, W_sage_r,
                    b_sage.reshape(1, HID), W_c1, b_c1.reshape(1, HID // 2),
                    W_c2p, bc2p)
    return out8[:, :1]
